# Initial kernel scaffold; baseline (speedup 1.0000x reference)
import functools

import jax
import jax.numpy as jnp
from jax.experimental import pallas as pl
from jax.experimental.pallas import tpu as pltpu
from jax.experimental.pallas import tpu_sc as plsc

N = 10000
E = 320000
H = 128
NC = 2
NS = 16
NW = NC * NS
LANES = 16
NPAD = 10240
ZB = NPAD // NS
CH = 128
NCHUNKS = E // CH
R = 2000

_F32 = jnp.float32


def _attn(h, asv_ref, adv_ref, as_ref, ad_ref):
    as_ref[...] = jnp.sum(h * asv_ref[...], axis=1)[None, :]
    ad_ref[...] = jnp.sum(h * adv_ref[...], axis=1)[None, :]


def _head_body(x_ref, w_ref, asv_ref, adv_ref, h_ref, as_ref, ad_ref):
    h = jnp.dot(x_ref[...], w_ref[...], preferred_element_type=_F32)
    h_ref[...] = h
    _attn(h, asv_ref, adv_ref, as_ref, ad_ref)


def _mid_self_body(parts_ref, b_ref, w_ref, asv_ref, adv_ref,
                   f_ref, h_ref, as_ref, ad_ref):
    f = parts_ref[0] + parts_ref[1] + b_ref[...]
    f_ref[...] = f
    ws = w_ref[0:H, :] + w_ref[H:2 * H, :]
    h = jnp.dot(f, ws, preferred_element_type=_F32)
    h_ref[...] = h
    _attn(h, asv_ref, adv_ref, as_ref, ad_ref)


def _mid_body(parts_ref, b_ref, g_ref, w_ref, asv_ref, adv_ref,
              f_ref, h_ref, as_ref, ad_ref):
    f = parts_ref[0] + parts_ref[1] + b_ref[...]
    f_ref[...] = f
    h = (jnp.dot(f, w_ref[0:H, :], preferred_element_type=_F32)
         + jnp.dot(g_ref[...], w_ref[H:2 * H, :], preferred_element_type=_F32))
    h_ref[...] = h
    _attn(h, asv_ref, adv_ref, as_ref, ad_ref)


def _mid_single_body(parts_ref, b_ref, w_ref, asv_ref, adv_ref,
                     h_ref, as_ref, ad_ref):
    f = parts_ref[0] + parts_ref[1] + b_ref[...]
    h = jnp.dot(f, w_ref[...], preferred_element_type=_F32)
    h_ref[...] = h
    _attn(h, asv_ref, adv_ref, as_ref, ad_ref)


def _tail_body(parts_ref, b_ref, out_ref):
    out_ref[...] = parts_ref[0] + parts_ref[1] + b_ref[...]


_vec_spec = pl.BlockSpec((1, H), lambda i: (0, 0))
_w1_spec = pl.BlockSpec((H, H), lambda i: (0, 0))
_w2_spec = pl.BlockSpec((2 * H, H), lambda i: (0, 0))
_row_spec = pl.BlockSpec((R, H), lambda i: (i, 0))
_parts_spec = pl.BlockSpec((2, R, H), lambda i: (0, i, 0))
_alpha_spec = pl.BlockSpec((1, R), lambda i: (0, i))

_h_shape = jax.ShapeDtypeStruct((N, H), _F32)
_alpha_shape = jax.ShapeDtypeStruct((1, N), _F32)


def _tc_head(x, w, asv, adv):
    return pl.pallas_call(
        _head_body,
        grid=(N // R,),
        in_specs=[_row_spec, _w1_spec, _vec_spec, _vec_spec],
        out_specs=[_row_spec, _alpha_spec, _alpha_spec],
        out_shape=[_h_shape, _alpha_shape, _alpha_shape],
    )(x, w, asv, adv)


def _tc_mid_self(parts, b, w, asv, adv):
    return pl.pallas_call(
        _mid_self_body,
        grid=(N // R,),
        in_specs=[_parts_spec, _vec_spec, _w2_spec, _vec_spec, _vec_spec],
        out_specs=[_row_spec, _row_spec, _alpha_spec, _alpha_spec],
        out_shape=[_h_shape, _h_shape, _alpha_shape, _alpha_shape],
    )(parts, b, w, asv, adv)


def _tc_mid(parts, b, g, w, asv, adv):
    return pl.pallas_call(
        _mid_body,
        grid=(N // R,),
        in_specs=[_parts_spec, _vec_spec, _row_spec, _w2_spec, _vec_spec,
                  _vec_spec],
        out_specs=[_row_spec, _row_spec, _alpha_spec, _alpha_spec],
        out_shape=[_h_shape, _h_shape, _alpha_shape, _alpha_shape],
    )(parts, b, g, w, asv, adv)


def _tc_mid_single(parts, b, w, asv, adv):
    return pl.pallas_call(
        _mid_single_body,
        grid=(N // R,),
        in_specs=[_parts_spec, _vec_spec, _w1_spec, _vec_spec, _vec_spec],
        out_specs=[_row_spec, _alpha_spec, _alpha_spec],
        out_shape=[_h_shape, _alpha_shape, _alpha_shape],
    )(parts, b, w, asv, adv)


def _tc_tail(parts, b):
    return pl.pallas_call(
        _tail_body,
        grid=(N // R,),
        in_specs=[_parts_spec, _vec_spec],
        out_specs=_row_spec,
        out_shape=_h_shape,
    )(parts, b)


def _mesh():
    return plsc.VectorSubcoreMesh(core_axis_name="c", subcore_axis_name="s")


def _worker_id():
    return jax.lax.axis_index("s") * NC + jax.lax.axis_index("c")


def _num_chunks(w):
    base = NCHUNKS // NW
    return jnp.where(w < NCHUNKS % NW, base + 1, base)


def _sc_pass1(edge_index, a_s, a_d):
    @functools.partial(
        pl.kernel,
        out_type=(jax.ShapeDtypeStruct((E,), _F32),
                  jax.ShapeDtypeStruct((NC, NPAD), _F32)),
        mesh=_mesh(),
        scratch_types=[
            pltpu.VMEM((N,), _F32),
            pltpu.VMEM((N,), _F32),
            pltpu.VMEM((CH,), jnp.int32),
            pltpu.VMEM((CH,), jnp.int32),
            pltpu.VMEM((CH,), _F32),
            pltpu.VMEM((ZB,), _F32),
            pltpu.VMEM_SHARED((NPAD,), _F32),
        ],
    )
    def k(ei_hbm, as_hbm, ad_hbm, p_hbm, den_hbm,
          as_v, ad_v, src_v, dst_v, p_v, z_v, den_sh):
        c = jax.lax.axis_index("c")
        s = jax.lax.axis_index("s")
        w = _worker_id()
        pltpu.sync_copy(as_hbm.at[0], as_v)
        pltpu.sync_copy(ad_hbm.at[0], ad_v)

        @pl.loop(0, ZB, step=LANES)
        def _(i):
            z_v[pl.ds(i, LANES)] = jnp.zeros((LANES,), _F32)

        pltpu.sync_copy(z_v, den_sh.at[pl.ds(s * ZB, ZB)])
        plsc.subcore_barrier()

        @pl.loop(0, _num_chunks(w))
        def _(k):
            base = (w + k * NW) * CH
            pltpu.sync_copy(ei_hbm.at[0, pl.ds(base, CH)], src_v)
            pltpu.sync_copy(ei_hbm.at[1, pl.ds(base, CH)], dst_v)

            @pl.loop(0, CH, step=LANES)
            def _(g):
                si = src_v[pl.ds(g, LANES)]
                di = dst_v[pl.ds(g, LANES)]
                e = plsc.load_gather(as_v, [si]) + plsc.load_gather(ad_v, [di])
                e = jnp.where(e >= 0.0, e, 0.2 * e)
                p_v[pl.ds(g, LANES)] = jnp.exp(e)

            pltpu.sync_copy(p_v, p_hbm.at[pl.ds(base, CH)])
            pltpu.sync_copy(p_v, den_sh.at[dst_v], add=True)

        plsc.subcore_barrier()
        pltpu.sync_copy(den_sh.at[pl.ds(s * ZB, ZB)],
                        den_hbm.at[c, pl.ds(s * ZB, ZB)])

    return k(edge_index, a_s, a_d)


def _sc_pass2(edge_index, p, den, h):
    @functools.partial(
        pl.kernel,
        out_type=jax.ShapeDtypeStruct((NC, NPAD, H), _F32),
        mesh=_mesh(),
        scratch_types=[
            pltpu.VMEM((NPAD,), _F32),
            pltpu.VMEM((NPAD,), _F32),
            pltpu.VMEM((CH,), jnp.int32),
            pltpu.VMEM((CH,), jnp.int32),
            pltpu.VMEM((CH,), _F32),
            pltpu.VMEM((CH,), _F32),
            pltpu.VMEM((CH, H), _F32),
            pltpu.VMEM_SHARED((NPAD, H), _F32),
        ],
    )
    def k(ei_hbm, p_hbm, den_hbm, h_hbm, out_hbm,
          invden_v, tmp_v, src_v, dst_v, p_v, w_v, rows_v, out_sh):
        c = jax.lax.axis_index("c")
        s = jax.lax.axis_index("s")
        w = _worker_id()
        pltpu.sync_copy(den_hbm.at[0], invden_v)
        pltpu.sync_copy(den_hbm.at[1], tmp_v)

        @pl.loop(0, NPAD, step=LANES)
        def _(i):
            d = invden_v[pl.ds(i, LANES)] + tmp_v[pl.ds(i, LANES)]
            invden_v[pl.ds(i, LANES)] = 1.0 / (d + 1e-16)

        @pl.loop(0, CH)
        def _(r):
            @pl.loop(0, H, step=LANES)
            def _(j):
                rows_v[r, pl.ds(j, LANES)] = jnp.zeros((LANES,), _F32)

        @pl.loop(0, ZB // CH)
        def _(j):
            pltpu.sync_copy(rows_v, out_sh.at[pl.ds(s * ZB + j * CH, CH)])

        plsc.subcore_barrier()

        @pl.loop(0, _num_chunks(w))
        def _(k):
            base = (w + k * NW) * CH
            pltpu.sync_copy(ei_hbm.at[0, pl.ds(base, CH)], src_v)
            pltpu.sync_copy(ei_hbm.at[1, pl.ds(base, CH)], dst_v)
            pltpu.sync_copy(p_hbm.at[pl.ds(base, CH)], p_v)
            pltpu.sync_copy(h_hbm.at[src_v], rows_v)

            @pl.loop(0, CH, step=LANES)
            def _(g):
                di = dst_v[pl.ds(g, LANES)]
                w_v[pl.ds(g, LANES)] = (p_v[pl.ds(g, LANES)]
                                        * plsc.load_gather(invden_v, [di]))

            @pl.loop(0, CH)
            def _(r):
                wsc = w_v[r]

                @pl.loop(0, H, step=LANES)
                def _(j):
                    rows_v[r, pl.ds(j, LANES)] = rows_v[r, pl.ds(j, LANES)] * wsc

            pltpu.sync_copy(rows_v, out_sh.at[dst_v], add=True)

        plsc.subcore_barrier()

        @pl.loop(0, ZB // CH)
        def _(j):
            pltpu.sync_copy(out_sh.at[pl.ds(s * ZB + j * CH, CH)],
                            out_hbm.at[c, pl.ds(s * ZB + j * CH, CH)])

    return k(edge_index, p, den, h)


def _gat_edges(edge_index, h, a_s, a_d):
    p, den = _sc_pass1(edge_index, a_s, a_d)
    return _sc_pass2(edge_index, p, den, h)


def kernel(x, edge_index, W_enc, a_src_enc, a_dst_enc, b_enc,
           W_proc, a_src_proc, a_dst_proc, b_proc,
           W_dec, a_src_dec, a_dst_dec, b_dec):
    ase, ade = a_src_enc.reshape(1, H), a_dst_enc.reshape(1, H)
    asp, adp = a_src_proc.reshape(1, H), a_dst_proc.reshape(1, H)
    asd, add = a_src_dec.reshape(1, H), a_dst_dec.reshape(1, H)
    be, bp, bd = b_enc.reshape(1, H), b_proc.reshape(1, H), b_dec.reshape(1, H)

    h1, as1, ad1 = _tc_head(x, W_enc, ase, ade)
    parts1 = _gat_edges(edge_index, h1, as1, ad1)
    enc, h2, as2, ad2 = _tc_mid_self(parts1, be, W_proc, asp, adp)
    parts2 = _gat_edges(edge_index, h2, as2, ad2)
    _, h3, as3, ad3 = _tc_mid(parts2, bp, enc, W_proc, asp, adp)
    parts3 = _gat_edges(edge_index, h3, as3, ad3)
    h4, as4, ad4 = _tc_mid_single(parts3, bp, W_dec, asd, add)
    parts4 = _gat_edges(edge_index, h4, as4, ad4)
    return _tc_tail(parts4, bd)


# speedup vs baseline: 21.6127x; 21.6127x over previous
import functools

import jax
import jax.numpy as jnp
from jax.experimental import pallas as pl
from jax.experimental.pallas import tpu as pltpu
from jax.experimental.pallas import tpu_sc as plsc

N = 10000
E = 320000
H = 128
NC = 2
NS = 16
NW = NC * NS
LANES = 16
NPAD = 10240
ZB = NPAD // NS
CH = 128
NCHUNKS = E // CH
R = 2000

_F32 = jnp.float32


def _attn(h, asv_ref, adv_ref, as_ref, ad_ref):
    as_ref[...] = jnp.sum(h * asv_ref[...], axis=1)[:, None]
    ad_ref[...] = jnp.sum(h * adv_ref[...], axis=1)[:, None]


def _head_body(x_ref, w_ref, asv_ref, adv_ref, h_ref, as_ref, ad_ref):
    h = jnp.dot(x_ref[...], w_ref[...], preferred_element_type=_F32)
    h_ref[...] = h
    _attn(h, asv_ref, adv_ref, as_ref, ad_ref)


def _mid_self_body(parts_ref, b_ref, w_ref, asv_ref, adv_ref,
                   f_ref, h_ref, as_ref, ad_ref):
    f = parts_ref[0] + parts_ref[1] + b_ref[...]
    f_ref[...] = f
    ws = w_ref[0:H, :] + w_ref[H:2 * H, :]
    h = jnp.dot(f, ws, preferred_element_type=_F32)
    h_ref[...] = h
    _attn(h, asv_ref, adv_ref, as_ref, ad_ref)


def _mid_body(parts_ref, b_ref, g_ref, w_ref, asv_ref, adv_ref,
              f_ref, h_ref, as_ref, ad_ref):
    f = parts_ref[0] + parts_ref[1] + b_ref[...]
    f_ref[...] = f
    h = (jnp.dot(f, w_ref[0:H, :], preferred_element_type=_F32)
         + jnp.dot(g_ref[...], w_ref[H:2 * H, :], preferred_element_type=_F32))
    h_ref[...] = h
    _attn(h, asv_ref, adv_ref, as_ref, ad_ref)


def _mid_single_body(parts_ref, b_ref, w_ref, asv_ref, adv_ref,
                     h_ref, as_ref, ad_ref):
    f = parts_ref[0] + parts_ref[1] + b_ref[...]
    h = jnp.dot(f, w_ref[...], preferred_element_type=_F32)
    h_ref[...] = h
    _attn(h, asv_ref, adv_ref, as_ref, ad_ref)


def _tail_body(parts_ref, b_ref, out_ref):
    out_ref[...] = parts_ref[0] + parts_ref[1] + b_ref[...]


_vec_spec = pl.BlockSpec((1, H), lambda i: (0, 0))
_w1_spec = pl.BlockSpec((H, H), lambda i: (0, 0))
_w2_spec = pl.BlockSpec((2 * H, H), lambda i: (0, 0))
_row_spec = pl.BlockSpec((R, H), lambda i: (i, 0))
_parts_spec = pl.BlockSpec((2, R, H), lambda i: (0, i, 0))
_alpha_spec = pl.BlockSpec((R, 1), lambda i: (i, 0))

_h_shape = jax.ShapeDtypeStruct((N, H), _F32)
_alpha_shape = jax.ShapeDtypeStruct((N, 1), _F32)


def _tc_head(x, w, asv, adv):
    return pl.pallas_call(
        _head_body,
        grid=(N // R,),
        in_specs=[_row_spec, _w1_spec, _vec_spec, _vec_spec],
        out_specs=[_row_spec, _alpha_spec, _alpha_spec],
        out_shape=[_h_shape, _alpha_shape, _alpha_shape],
    )(x, w, asv, adv)


def _tc_mid_self(parts, b, w, asv, adv):
    return pl.pallas_call(
        _mid_self_body,
        grid=(N // R,),
        in_specs=[_parts_spec, _vec_spec, _w2_spec, _vec_spec, _vec_spec],
        out_specs=[_row_spec, _row_spec, _alpha_spec, _alpha_spec],
        out_shape=[_h_shape, _h_shape, _alpha_shape, _alpha_shape],
    )(parts, b, w, asv, adv)


def _tc_mid(parts, b, g, w, asv, adv):
    return pl.pallas_call(
        _mid_body,
        grid=(N // R,),
        in_specs=[_parts_spec, _vec_spec, _row_spec, _w2_spec, _vec_spec,
                  _vec_spec],
        out_specs=[_row_spec, _row_spec, _alpha_spec, _alpha_spec],
        out_shape=[_h_shape, _h_shape, _alpha_shape, _alpha_shape],
    )(parts, b, g, w, asv, adv)


def _tc_mid_single(parts, b, w, asv, adv):
    return pl.pallas_call(
        _mid_single_body,
        grid=(N // R,),
        in_specs=[_parts_spec, _vec_spec, _w1_spec, _vec_spec, _vec_spec],
        out_specs=[_row_spec, _alpha_spec, _alpha_spec],
        out_shape=[_h_shape, _alpha_shape, _alpha_shape],
    )(parts, b, w, asv, adv)


def _tc_tail(parts, b):
    return pl.pallas_call(
        _tail_body,
        grid=(N // R,),
        in_specs=[_parts_spec, _vec_spec],
        out_specs=_row_spec,
        out_shape=_h_shape,
    )(parts, b)


def _mesh():
    return plsc.VectorSubcoreMesh(core_axis_name="c", subcore_axis_name="s")


_SC_PARAMS = pltpu.CompilerParams(needs_layout_passes=False)


def _worker_id():
    return jax.lax.axis_index("s") * NC + jax.lax.axis_index("c")


def _num_chunks(w):
    base = NCHUNKS // NW
    return jnp.where(w < NCHUNKS % NW, base + 1, base)


def _sc_pass1(edge_index, a_s, a_d):
    @functools.partial(
        pl.kernel,
        out_type=(jax.ShapeDtypeStruct((E,), _F32),
                  jax.ShapeDtypeStruct((NC, NPAD), _F32)),
        mesh=_mesh(),
        scratch_types=[
            pltpu.VMEM((N,), _F32),
            pltpu.VMEM((N,), _F32),
            pltpu.VMEM((CH,), jnp.int32),
            pltpu.VMEM((CH,), jnp.int32),
            pltpu.VMEM((CH,), _F32),
            pltpu.VMEM((ZB,), _F32),
            pltpu.VMEM_SHARED((NPAD,), _F32),
        ],
        compiler_params=_SC_PARAMS,
    )
    def k(ei_hbm, as_hbm, ad_hbm, p_hbm, den_hbm,
          as_v, ad_v, src_v, dst_v, p_v, z_v, den_sh):
        c = jax.lax.axis_index("c")
        s = jax.lax.axis_index("s")
        w = _worker_id()
        pltpu.sync_copy(as_hbm, as_v)
        pltpu.sync_copy(ad_hbm, ad_v)

        @pl.loop(0, ZB, step=LANES)
        def _(i):
            z_v[pl.ds(i, LANES)] = jnp.zeros((LANES,), _F32)

        pltpu.sync_copy(z_v, den_sh.at[pl.ds(s * ZB, ZB)])
        plsc.subcore_barrier()

        @pl.loop(0, _num_chunks(w))
        def _(k):
            base = (w + k * NW) * CH
            pltpu.sync_copy(ei_hbm.at[0, pl.ds(base, CH)], src_v)
            pltpu.sync_copy(ei_hbm.at[1, pl.ds(base, CH)], dst_v)

            @pl.loop(0, CH, step=LANES)
            def _(g):
                si = src_v[pl.ds(g, LANES)]
                di = dst_v[pl.ds(g, LANES)]
                e = plsc.load_gather(as_v, [si]) + plsc.load_gather(ad_v, [di])
                e = jnp.where(e >= 0.0, e, 0.2 * e)
                p_v[pl.ds(g, LANES)] = jnp.exp(e)

            pltpu.sync_copy(p_v, p_hbm.at[pl.ds(base, CH)])
            pltpu.sync_copy(p_v, den_sh.at[dst_v], add=True)

        plsc.subcore_barrier()
        pltpu.sync_copy(den_sh.at[pl.ds(s * ZB, ZB)],
                        den_hbm.at[c, pl.ds(s * ZB, ZB)])

    return k(edge_index, a_s, a_d)


def _sc_pass2(edge_index, p, den, h):
    @functools.partial(
        pl.kernel,
        out_type=jax.ShapeDtypeStruct((NC, NPAD, H), _F32),
        mesh=_mesh(),
        scratch_types=[
            pltpu.VMEM((NPAD,), _F32),
            pltpu.VMEM((NPAD,), _F32),
            pltpu.VMEM((CH,), jnp.int32),
            pltpu.VMEM((CH,), jnp.int32),
            pltpu.VMEM((CH,), _F32),
            pltpu.VMEM((CH,), _F32),
            pltpu.VMEM((CH, H), _F32),
            pltpu.VMEM_SHARED((NPAD, H), _F32),
        ],
        compiler_params=_SC_PARAMS,
    )
    def k(ei_hbm, p_hbm, den_hbm, h_hbm, out_hbm,
          invden_v, tmp_v, src_v, dst_v, p_v, w_v, rows_v, out_sh):
        c = jax.lax.axis_index("c")
        s = jax.lax.axis_index("s")
        w = _worker_id()
        pltpu.sync_copy(den_hbm.at[0], invden_v)
        pltpu.sync_copy(den_hbm.at[1], tmp_v)

        @pl.loop(0, NPAD, step=LANES)
        def _(i):
            d = invden_v[pl.ds(i, LANES)] + tmp_v[pl.ds(i, LANES)]
            invden_v[pl.ds(i, LANES)] = 1.0 / (d + 1e-16)

        @pl.loop(0, CH)
        def _(r):
            @pl.loop(0, H, step=LANES)
            def _(j):
                rows_v[r, pl.ds(j, LANES)] = jnp.zeros((LANES,), _F32)

        @pl.loop(0, ZB // CH)
        def _(j):
            pltpu.sync_copy(rows_v, out_sh.at[pl.ds(s * ZB + j * CH, CH)])

        plsc.subcore_barrier()

        @pl.loop(0, _num_chunks(w))
        def _(k):
            base = (w + k * NW) * CH
            pltpu.sync_copy(ei_hbm.at[0, pl.ds(base, CH)], src_v)
            pltpu.sync_copy(ei_hbm.at[1, pl.ds(base, CH)], dst_v)
            pltpu.sync_copy(p_hbm.at[pl.ds(base, CH)], p_v)
            pltpu.sync_copy(h_hbm.at[src_v], rows_v)

            @pl.loop(0, CH, step=LANES)
            def _(g):
                di = dst_v[pl.ds(g, LANES)]
                w_v[pl.ds(g, LANES)] = (p_v[pl.ds(g, LANES)]
                                        * plsc.load_gather(invden_v, [di]))

            @pl.loop(0, CH, step=LANES)
            def _(g):
                w16 = w_v[pl.ds(g, LANES)]
                for j in range(LANES):
                    wj = w16[j]
                    for cb in range(H // LANES):
                        cs = pl.ds(cb * LANES, LANES)
                        rows_v[g + j, cs] = rows_v[g + j, cs] * wj

            pltpu.sync_copy(rows_v, out_sh.at[dst_v], add=True)

        plsc.subcore_barrier()

        @pl.loop(0, ZB // CH)
        def _(j):
            pltpu.sync_copy(out_sh.at[pl.ds(s * ZB + j * CH, CH)],
                            out_hbm.at[c, pl.ds(s * ZB + j * CH, CH)])

    return k(edge_index, p, den, h)


def _gat_edges(edge_index, h, a_s, a_d):
    p, den = _sc_pass1(edge_index, a_s.reshape(N), a_d.reshape(N))
    return _sc_pass2(edge_index, p, den, h)


def kernel(x, edge_index, W_enc, a_src_enc, a_dst_enc, b_enc,
           W_proc, a_src_proc, a_dst_proc, b_proc,
           W_dec, a_src_dec, a_dst_dec, b_dec):
    ase, ade = a_src_enc.reshape(1, H), a_dst_enc.reshape(1, H)
    asp, adp = a_src_proc.reshape(1, H), a_dst_proc.reshape(1, H)
    asd, add = a_src_dec.reshape(1, H), a_dst_dec.reshape(1, H)
    be, bp, bd = b_enc.reshape(1, H), b_proc.reshape(1, H), b_dec.reshape(1, H)

    h1, as1, ad1 = _tc_head(x, W_enc, ase, ade)
    parts1 = _gat_edges(edge_index, h1, as1, ad1)
    enc, h2, as2, ad2 = _tc_mid_self(parts1, be, W_proc, asp, adp)
    parts2 = _gat_edges(edge_index, h2, as2, ad2)
    _, h3, as3, ad3 = _tc_mid(parts2, bp, enc, W_proc, asp, adp)
    parts3 = _gat_edges(edge_index, h3, as3, ad3)
    h4, as4, ad4 = _tc_mid_single(parts3, bp, W_dec, asd, add)
    parts4 = _gat_edges(edge_index, h4, as4, ad4)
    return _tc_tail(parts4, bd)


# speedup vs baseline: 34.6838x; 1.6048x over previous
import functools

import jax
import jax.numpy as jnp
from jax.experimental import pallas as pl
from jax.experimental.pallas import tpu as pltpu
from jax.experimental.pallas import tpu_sc as plsc

N = 10000
E = 320000
H = 128
NC = 2
NS = 16
NW = NC * NS
LANES = 16
NPAD = 10240
ZB = NPAD // NS
CH = 128
NCHUNKS = E // CH
R = 2000

_F32 = jnp.float32


def _attn(h, asv_ref, adv_ref, as_ref, ad_ref):
    as_ref[...] = jnp.sum(h * asv_ref[...], axis=1)[:, None]
    ad_ref[...] = jnp.sum(h * adv_ref[...], axis=1)[:, None]


def _head_body(x_ref, w_ref, asv_ref, adv_ref, h_ref, as_ref, ad_ref):
    h = jnp.dot(x_ref[...], w_ref[...], preferred_element_type=_F32)
    h_ref[...] = h
    _attn(h, asv_ref, adv_ref, as_ref, ad_ref)


def _mid_self_body(parts_ref, b_ref, w_ref, asv_ref, adv_ref,
                   f_ref, h_ref, as_ref, ad_ref):
    f = parts_ref[0] + parts_ref[1] + b_ref[...]
    f_ref[...] = f
    ws = w_ref[0:H, :] + w_ref[H:2 * H, :]
    h = jnp.dot(f, ws, preferred_element_type=_F32)
    h_ref[...] = h
    _attn(h, asv_ref, adv_ref, as_ref, ad_ref)


def _mid_body(parts_ref, b_ref, g_ref, w_ref, asv_ref, adv_ref,
              f_ref, h_ref, as_ref, ad_ref):
    f = parts_ref[0] + parts_ref[1] + b_ref[...]
    f_ref[...] = f
    h = (jnp.dot(f, w_ref[0:H, :], preferred_element_type=_F32)
         + jnp.dot(g_ref[...], w_ref[H:2 * H, :], preferred_element_type=_F32))
    h_ref[...] = h
    _attn(h, asv_ref, adv_ref, as_ref, ad_ref)


def _mid_single_body(parts_ref, b_ref, w_ref, asv_ref, adv_ref,
                     h_ref, as_ref, ad_ref):
    f = parts_ref[0] + parts_ref[1] + b_ref[...]
    h = jnp.dot(f, w_ref[...], preferred_element_type=_F32)
    h_ref[...] = h
    _attn(h, asv_ref, adv_ref, as_ref, ad_ref)


def _tail_body(parts_ref, b_ref, out_ref):
    out_ref[...] = parts_ref[0] + parts_ref[1] + b_ref[...]


def _invden_body(den_ref, out_ref):
    out_ref[...] = (1.0 / (den_ref[0] + den_ref[1] + 1e-16))[None, :]


_vec_spec = pl.BlockSpec((1, H), lambda i: (0, 0))
_w1_spec = pl.BlockSpec((H, H), lambda i: (0, 0))
_w2_spec = pl.BlockSpec((2 * H, H), lambda i: (0, 0))
_row_spec = pl.BlockSpec((R, H), lambda i: (i, 0))
_parts_spec = pl.BlockSpec((2, R, H), lambda i: (0, i, 0))
_alpha_spec = pl.BlockSpec((R, 1), lambda i: (i, 0))

_h_shape = jax.ShapeDtypeStruct((N, H), _F32)
_alpha_shape = jax.ShapeDtypeStruct((N, 1), _F32)


def _tc_head(x, w, asv, adv):
    return pl.pallas_call(
        _head_body,
        grid=(N // R,),
        in_specs=[_row_spec, _w1_spec, _vec_spec, _vec_spec],
        out_specs=[_row_spec, _alpha_spec, _alpha_spec],
        out_shape=[_h_shape, _alpha_shape, _alpha_shape],
    )(x, w, asv, adv)


def _tc_mid_self(parts, b, w, asv, adv):
    return pl.pallas_call(
        _mid_self_body,
        grid=(N // R,),
        in_specs=[_parts_spec, _vec_spec, _w2_spec, _vec_spec, _vec_spec],
        out_specs=[_row_spec, _row_spec, _alpha_spec, _alpha_spec],
        out_shape=[_h_shape, _h_shape, _alpha_shape, _alpha_shape],
    )(parts, b, w, asv, adv)


def _tc_mid(parts, b, g, w, asv, adv):
    return pl.pallas_call(
        _mid_body,
        grid=(N // R,),
        in_specs=[_parts_spec, _vec_spec, _row_spec, _w2_spec, _vec_spec,
                  _vec_spec],
        out_specs=[_row_spec, _row_spec, _alpha_spec, _alpha_spec],
        out_shape=[_h_shape, _h_shape, _alpha_shape, _alpha_shape],
    )(parts, b, g, w, asv, adv)


def _tc_mid_single(parts, b, w, asv, adv):
    return pl.pallas_call(
        _mid_single_body,
        grid=(N // R,),
        in_specs=[_parts_spec, _vec_spec, _w1_spec, _vec_spec, _vec_spec],
        out_specs=[_row_spec, _alpha_spec, _alpha_spec],
        out_shape=[_h_shape, _alpha_shape, _alpha_shape],
    )(parts, b, w, asv, adv)


def _tc_invden(den):
    return pl.pallas_call(
        _invden_body,
        in_specs=[pl.BlockSpec((2, NPAD), lambda: (0, 0))],
        out_specs=pl.BlockSpec((1, NPAD), lambda: (0, 0)),
        out_shape=jax.ShapeDtypeStruct((1, NPAD), _F32),
    )(den)


def _tc_tail(parts, b):
    return pl.pallas_call(
        _tail_body,
        grid=(N // R,),
        in_specs=[_parts_spec, _vec_spec],
        out_specs=_row_spec,
        out_shape=_h_shape,
    )(parts, b)


def _mesh():
    return plsc.VectorSubcoreMesh(core_axis_name="c", subcore_axis_name="s")


_SC_PARAMS = pltpu.CompilerParams(needs_layout_passes=False)


def _worker_id():
    return jax.lax.axis_index("s") * NC + jax.lax.axis_index("c")


PT = E // NW
NK = PT // CH
REM = PT - NK * CH


def _sc_pass1(src, dst, a_s, a_d):
    @functools.partial(
        pl.kernel,
        out_type=(jax.ShapeDtypeStruct((E,), _F32),
                  jax.ShapeDtypeStruct((NC, NPAD), _F32)),
        mesh=_mesh(),
        scratch_types=[
            pltpu.VMEM((N,), _F32),
            pltpu.VMEM((N,), _F32),
            pltpu.VMEM((2, CH), jnp.int32),
            pltpu.VMEM((2, CH), jnp.int32),
            pltpu.VMEM((2, CH), _F32),
            pltpu.VMEM((ZB,), _F32),
            pltpu.VMEM((REM,), jnp.int32),
            pltpu.VMEM_SHARED((NPAD,), _F32),
            pltpu.SemaphoreType.DMA,
            pltpu.SemaphoreType.DMA,
            pltpu.SemaphoreType.DMA,
            pltpu.SemaphoreType.DMA,
        ],
        compiler_params=_SC_PARAMS,
    )
    def k(src_hbm, dst_hbm, as_hbm, ad_hbm, p_hbm, den_hbm,
          as_v, ad_v, src_v, dst_v, p_v, z_v, rdst_v, den_sh,
          sl0, sl1, so0, so1):
        c = jax.lax.axis_index("c")
        s = jax.lax.axis_index("s")
        w = _worker_id()
        base_w = w * PT
        sl = (sl0, sl1)
        so = (so0, so1)
        pltpu.sync_copy(as_hbm, as_v)
        pltpu.sync_copy(ad_hbm, ad_v)

        @pl.loop(0, ZB, step=LANES)
        def _(i):
            z_v[pl.ds(i, LANES)] = jnp.zeros((LANES,), _F32)

        pltpu.sync_copy(z_v, den_sh.at[pl.ds(s * ZB, ZB)])
        plsc.subcore_barrier()

        def issue_load(k, b):
            base = base_w + k * CH
            pltpu.async_copy(src_hbm.at[pl.ds(base, CH)], src_v.at[b], sl[b])
            pltpu.async_copy(dst_hbm.at[pl.ds(base, CH)], dst_v.at[b], sl[b])

        def wait_load(k, b):
            base = base_w + k * CH
            pltpu.make_async_copy(src_hbm.at[pl.ds(base, CH)], src_v.at[b],
                                  sl[b]).wait()
            pltpu.make_async_copy(dst_hbm.at[pl.ds(base, CH)], dst_v.at[b],
                                  sl[b]).wait()

        def wait_out(k, b):
            base = base_w + k * CH
            pltpu.make_async_copy(p_v.at[b], p_hbm.at[pl.ds(base, CH)],
                                  so[b]).wait()

        def compute(b):
            @pl.loop(0, CH, step=LANES)
            def _(g):
                si = src_v[b, pl.ds(g, LANES)]
                di = dst_v[b, pl.ds(g, LANES)]
                e = plsc.load_gather(as_v, [si]) + plsc.load_gather(ad_v, [di])
                e = jnp.where(e >= 0.0, e, 0.2 * e)
                p_v[b, pl.ds(g, LANES)] = jnp.exp(e)

        def step(k, b):
            @pl.when(k > 1)
            def _():
                wait_out(k - 2, b)

            compute(b)
            base = base_w + k * CH
            pltpu.async_copy(p_v.at[b], p_hbm.at[pl.ds(base, CH)], so[b])
            pltpu.sync_copy(p_v.at[b], den_sh.at[dst_v.at[b]], add=True)

            @pl.when(k + 2 < NK)
            def _():
                issue_load(k + 2, b)

            @pl.when(k + 1 < NK)
            def _():
                wait_load(k + 1, 1 - b)

        pltpu.sync_copy(src_hbm.at[pl.ds(base_w, CH)], src_v.at[0])
        pltpu.sync_copy(dst_hbm.at[pl.ds(base_w, CH)], dst_v.at[0])
        issue_load(1, 1)

        @pl.loop(0, NK // 2)
        def _(kp):
            step(2 * kp, 0)
            step(2 * kp + 1, 1)

        wait_out(NK - 2, 0)
        wait_out(NK - 1, 1)

        rbase = base_w + NK * CH
        pltpu.sync_copy(src_hbm.at[pl.ds(rbase, REM)],
                        src_v.at[0, pl.ds(0, REM)])
        pltpu.sync_copy(dst_hbm.at[pl.ds(rbase, REM)],
                        dst_v.at[0, pl.ds(0, REM)])
        si = src_v[0, pl.ds(0, LANES)]
        di = dst_v[0, pl.ds(0, LANES)]
        rdst_v[...] = di
        e = plsc.load_gather(as_v, [si]) + plsc.load_gather(ad_v, [di])
        e = jnp.where(e >= 0.0, e, 0.2 * e)
        p_v[0, pl.ds(0, LANES)] = jnp.exp(e)
        pltpu.sync_copy(p_v.at[0, pl.ds(0, REM)], p_hbm.at[pl.ds(rbase, REM)])
        pltpu.sync_copy(p_v.at[0, pl.ds(0, REM)], den_sh.at[rdst_v], add=True)

        plsc.subcore_barrier()
        pltpu.sync_copy(den_sh.at[pl.ds(s * ZB, ZB)],
                        den_hbm.at[c, pl.ds(s * ZB, ZB)])

    return k(src, dst, a_s, a_d)


def _sc_pass2(src, dst, p, invden, h):
    @functools.partial(
        pl.kernel,
        out_type=jax.ShapeDtypeStruct((NC, NPAD, H), _F32),
        mesh=_mesh(),
        scratch_types=[
            pltpu.VMEM((NPAD,), _F32),
            pltpu.VMEM((2, CH), jnp.int32),
            pltpu.VMEM((2, CH), jnp.int32),
            pltpu.VMEM((2, CH), _F32),
            pltpu.VMEM((REM,), jnp.int32),
            pltpu.VMEM((CH, H), _F32),
            pltpu.VMEM_SHARED((NPAD, H), _F32),
            pltpu.SemaphoreType.DMA,
            pltpu.SemaphoreType.DMA,
            pltpu.SemaphoreType.DMA,
        ],
        compiler_params=_SC_PARAMS,
    )
    def k(src_hbm, dst_hbm, p_hbm, invden_hbm, h_hbm, out_hbm,
          invden_v, src_v, dst_v, p_v, rdst_v, rows_v, out_sh,
          sl0, sl1, sg):
        c = jax.lax.axis_index("c")
        s = jax.lax.axis_index("s")
        w = _worker_id()
        base_w = w * PT
        sl = (sl0, sl1)
        pltpu.sync_copy(invden_hbm, invden_v)

        @pl.loop(0, CH)
        def _(r):
            @pl.loop(0, H, step=LANES)
            def _(j):
                rows_v[r, pl.ds(j, LANES)] = jnp.zeros((LANES,), _F32)

        @pl.loop(0, ZB // CH)
        def _(j):
            pltpu.sync_copy(rows_v, out_sh.at[pl.ds(s * ZB + j * CH, CH)])

        plsc.subcore_barrier()

        def issue_load(k, b):
            base = base_w + k * CH
            pltpu.async_copy(src_hbm.at[pl.ds(base, CH)], src_v.at[b], sl[b])
            pltpu.async_copy(dst_hbm.at[pl.ds(base, CH)], dst_v.at[b], sl[b])
            pltpu.async_copy(p_hbm.at[pl.ds(base, CH)], p_v.at[b], sl[b])

        def wait_load(k, b):
            base = base_w + k * CH
            pltpu.make_async_copy(src_hbm.at[pl.ds(base, CH)], src_v.at[b],
                                  sl[b]).wait()
            pltpu.make_async_copy(dst_hbm.at[pl.ds(base, CH)], dst_v.at[b],
                                  sl[b]).wait()
            pltpu.make_async_copy(p_hbm.at[pl.ds(base, CH)], p_v.at[b],
                                  sl[b]).wait()

        def step(k, b):
            gather = pltpu.async_copy(h_hbm.at[src_v.at[b]], rows_v, sg)

            @pl.loop(0, CH, step=LANES)
            def _(g):
                di = dst_v[b, pl.ds(g, LANES)]
                p_v[b, pl.ds(g, LANES)] = (p_v[b, pl.ds(g, LANES)]
                                           * plsc.load_gather(invden_v, [di]))

            gather.wait()

            @pl.loop(0, CH, step=LANES)
            def _(g):
                w16 = p_v[b, pl.ds(g, LANES)]
                for j in range(LANES):
                    wj = w16[j]
                    for cb in range(H // LANES):
                        cs = pl.ds(cb * LANES, LANES)
                        rows_v[g + j, cs] = rows_v[g + j, cs] * wj

            pltpu.sync_copy(rows_v, out_sh.at[dst_v.at[b]], add=True)

            @pl.when(k + 2 < NK)
            def _():
                issue_load(k + 2, b)

            @pl.when(k + 1 < NK)
            def _():
                wait_load(k + 1, 1 - b)

        pltpu.sync_copy(src_hbm.at[pl.ds(base_w, CH)], src_v.at[0])
        pltpu.sync_copy(dst_hbm.at[pl.ds(base_w, CH)], dst_v.at[0])
        pltpu.sync_copy(p_hbm.at[pl.ds(base_w, CH)], p_v.at[0])
        issue_load(1, 1)

        @pl.loop(0, NK // 2)
        def _(kp):
            step(2 * kp, 0)
            step(2 * kp + 1, 1)

        rbase = base_w + NK * CH
        pltpu.sync_copy(src_hbm.at[pl.ds(rbase, REM)],
                        src_v.at[0, pl.ds(0, REM)])
        pltpu.sync_copy(dst_hbm.at[pl.ds(rbase, REM)],
                        dst_v.at[0, pl.ds(0, REM)])
        pltpu.sync_copy(p_hbm.at[pl.ds(rbase, REM)], p_v.at[0, pl.ds(0, REM)])
        rsi = src_v[0, pl.ds(0, LANES)]
        rdi = dst_v[0, pl.ds(0, LANES)]
        rdst_v[...] = rsi
        pltpu.sync_copy(h_hbm.at[rdst_v], rows_v.at[pl.ds(0, REM)])
        rdst_v[...] = rdi
        w16 = p_v[0, pl.ds(0, LANES)] * plsc.load_gather(invden_v, [rdi])
        for j in range(LANES):
            wj = w16[j]
            for cb in range(H // LANES):
                cs = pl.ds(cb * LANES, LANES)
                rows_v[j, cs] = rows_v[j, cs] * wj
        pltpu.sync_copy(rows_v.at[pl.ds(0, REM)], out_sh.at[rdst_v],
                        add=True)

        plsc.subcore_barrier()

        @pl.loop(0, ZB // CH)
        def _(j):
            pltpu.sync_copy(out_sh.at[pl.ds(s * ZB + j * CH, CH)],
                            out_hbm.at[c, pl.ds(s * ZB + j * CH, CH)])

    return k(src, dst, p, invden, h)


def _gat_edges(src, dst, h, a_s, a_d):
    p, den = _sc_pass1(src, dst, a_s.reshape(N), a_d.reshape(N))
    invden = _tc_invden(den).reshape(NPAD)
    return _sc_pass2(src, dst, p, invden, h)


def kernel(x, edge_index, W_enc, a_src_enc, a_dst_enc, b_enc,
           W_proc, a_src_proc, a_dst_proc, b_proc,
           W_dec, a_src_dec, a_dst_dec, b_dec):
    ase, ade = a_src_enc.reshape(1, H), a_dst_enc.reshape(1, H)
    asp, adp = a_src_proc.reshape(1, H), a_dst_proc.reshape(1, H)
    asd, add = a_src_dec.reshape(1, H), a_dst_dec.reshape(1, H)
    be, bp, bd = b_enc.reshape(1, H), b_proc.reshape(1, H), b_dec.reshape(1, H)

    src, dst = edge_index[0], edge_index[1]
    h1, as1, ad1 = _tc_head(x, W_enc, ase, ade)
    parts1 = _gat_edges(src, dst, h1, as1, ad1)
    enc, h2, as2, ad2 = _tc_mid_self(parts1, be, W_proc, asp, adp)
    parts2 = _gat_edges(src, dst, h2, as2, ad2)
    _, h3, as3, ad3 = _tc_mid(parts2, bp, enc, W_proc, asp, adp)
    parts3 = _gat_edges(src, dst, h3, as3, ad3)
    h4, as4, ad4 = _tc_mid_single(parts3, bp, W_dec, asd, add)
    parts4 = _gat_edges(src, dst, h4, as4, ad4)
    return _tc_tail(parts4, bd)


# speedup vs baseline: 36.0530x; 1.0395x over previous
import functools

import jax
import jax.numpy as jnp
from jax.experimental import pallas as pl
from jax.experimental.pallas import tpu as pltpu
from jax.experimental.pallas import tpu_sc as plsc

N = 10000
E = 320000
H = 128
NC = 2
NS = 16
NW = NC * NS
LANES = 16
NPAD = 10240
ZB = NPAD // NS
CH = 128
NCHUNKS = E // CH
R = 2000

_F32 = jnp.float32


def _attn(h, asv_ref, adv_ref, as_ref, ad_ref):
    as_ref[...] = jnp.sum(h * asv_ref[...], axis=1)[:, None]
    ad_ref[...] = jnp.sum(h * adv_ref[...], axis=1)[:, None]


def _head_body(x_ref, w_ref, asv_ref, adv_ref, h_ref, as_ref, ad_ref):
    h = jnp.dot(x_ref[...], w_ref[...], preferred_element_type=_F32)
    h_ref[...] = h
    _attn(h, asv_ref, adv_ref, as_ref, ad_ref)


def _mid_self_body(parts_ref, b_ref, w_ref, asv_ref, adv_ref,
                   f_ref, h_ref, as_ref, ad_ref):
    f = parts_ref[0] + parts_ref[1] + b_ref[...]
    f_ref[...] = f
    ws = w_ref[0:H, :] + w_ref[H:2 * H, :]
    h = jnp.dot(f, ws, preferred_element_type=_F32)
    h_ref[...] = h
    _attn(h, asv_ref, adv_ref, as_ref, ad_ref)


def _mid_body(parts_ref, b_ref, g_ref, w_ref, asv_ref, adv_ref,
              f_ref, h_ref, as_ref, ad_ref):
    f = parts_ref[0] + parts_ref[1] + b_ref[...]
    f_ref[...] = f
    h = (jnp.dot(f, w_ref[0:H, :], preferred_element_type=_F32)
         + jnp.dot(g_ref[...], w_ref[H:2 * H, :], preferred_element_type=_F32))
    h_ref[...] = h
    _attn(h, asv_ref, adv_ref, as_ref, ad_ref)


def _mid_single_body(parts_ref, b_ref, w_ref, asv_ref, adv_ref,
                     h_ref, as_ref, ad_ref):
    f = parts_ref[0] + parts_ref[1] + b_ref[...]
    h = jnp.dot(f, w_ref[...], preferred_element_type=_F32)
    h_ref[...] = h
    _attn(h, asv_ref, adv_ref, as_ref, ad_ref)


def _tail_body(parts_ref, b_ref, out_ref):
    out_ref[...] = parts_ref[0] + parts_ref[1] + b_ref[...]


def _invden_body(den_ref, out_ref):
    out_ref[...] = (1.0 / (den_ref[0] + den_ref[1] + 1e-16))[None, :]


_vec_spec = pl.BlockSpec((1, H), lambda i: (0, 0))
_w1_spec = pl.BlockSpec((H, H), lambda i: (0, 0))
_w2_spec = pl.BlockSpec((2 * H, H), lambda i: (0, 0))
_row_spec = pl.BlockSpec((R, H), lambda i: (i, 0))
_parts_spec = pl.BlockSpec((2, R, H), lambda i: (0, i, 0))
_alpha_spec = pl.BlockSpec((R, 1), lambda i: (i, 0))

_h_shape = jax.ShapeDtypeStruct((N, H), _F32)
_alpha_shape = jax.ShapeDtypeStruct((N, 1), _F32)


def _tc_head(x, w, asv, adv):
    return pl.pallas_call(
        _head_body,
        grid=(N // R,),
        in_specs=[_row_spec, _w1_spec, _vec_spec, _vec_spec],
        out_specs=[_row_spec, _alpha_spec, _alpha_spec],
        out_shape=[_h_shape, _alpha_shape, _alpha_shape],
    )(x, w, asv, adv)


def _tc_mid_self(parts, b, w, asv, adv):
    return pl.pallas_call(
        _mid_self_body,
        grid=(N // R,),
        in_specs=[_parts_spec, _vec_spec, _w2_spec, _vec_spec, _vec_spec],
        out_specs=[_row_spec, _row_spec, _alpha_spec, _alpha_spec],
        out_shape=[_h_shape, _h_shape, _alpha_shape, _alpha_shape],
    )(parts, b, w, asv, adv)


def _tc_mid(parts, b, g, w, asv, adv):
    return pl.pallas_call(
        _mid_body,
        grid=(N // R,),
        in_specs=[_parts_spec, _vec_spec, _row_spec, _w2_spec, _vec_spec,
                  _vec_spec],
        out_specs=[_row_spec, _row_spec, _alpha_spec, _alpha_spec],
        out_shape=[_h_shape, _h_shape, _alpha_shape, _alpha_shape],
    )(parts, b, g, w, asv, adv)


def _tc_mid_single(parts, b, w, asv, adv):
    return pl.pallas_call(
        _mid_single_body,
        grid=(N // R,),
        in_specs=[_parts_spec, _vec_spec, _w1_spec, _vec_spec, _vec_spec],
        out_specs=[_row_spec, _alpha_spec, _alpha_spec],
        out_shape=[_h_shape, _alpha_shape, _alpha_shape],
    )(parts, b, w, asv, adv)


def _tc_invden(den):
    return pl.pallas_call(
        _invden_body,
        in_specs=[pl.BlockSpec((2, NPAD), lambda: (0, 0))],
        out_specs=pl.BlockSpec((1, NPAD), lambda: (0, 0)),
        out_shape=jax.ShapeDtypeStruct((1, NPAD), _F32),
    )(den)


def _tc_tail(parts, b):
    return pl.pallas_call(
        _tail_body,
        grid=(N // R,),
        in_specs=[_parts_spec, _vec_spec],
        out_specs=_row_spec,
        out_shape=_h_shape,
    )(parts, b)


def _mesh():
    return plsc.VectorSubcoreMesh(core_axis_name="c", subcore_axis_name="s")


_SC_PARAMS = pltpu.CompilerParams(needs_layout_passes=False)


def _worker_id():
    return jax.lax.axis_index("s") * NC + jax.lax.axis_index("c")


PT = E // NW
NK = PT // CH
REM = PT - NK * CH


def _sc_pass1(src, dst, a_s, a_d):
    @functools.partial(
        pl.kernel,
        out_type=(jax.ShapeDtypeStruct((E,), _F32),
                  jax.ShapeDtypeStruct((NC, NPAD), _F32)),
        mesh=_mesh(),
        scratch_types=[
            pltpu.VMEM((N,), _F32),
            pltpu.VMEM((N,), _F32),
            pltpu.VMEM((2, CH), jnp.int32),
            pltpu.VMEM((2, CH), jnp.int32),
            pltpu.VMEM((2, CH), _F32),
            pltpu.VMEM((ZB,), _F32),
            pltpu.VMEM((REM,), jnp.int32),
            pltpu.VMEM_SHARED((NPAD,), _F32),
            pltpu.SemaphoreType.DMA,
            pltpu.SemaphoreType.DMA,
            pltpu.SemaphoreType.DMA,
            pltpu.SemaphoreType.DMA,
        ],
        compiler_params=_SC_PARAMS,
    )
    def k(src_hbm, dst_hbm, as_hbm, ad_hbm, p_hbm, den_hbm,
          as_v, ad_v, src_v, dst_v, p_v, z_v, rdst_v, den_sh,
          sl0, sl1, so0, so1):
        c = jax.lax.axis_index("c")
        s = jax.lax.axis_index("s")
        w = _worker_id()
        base_w = w * PT
        sl = (sl0, sl1)
        so = (so0, so1)
        pltpu.sync_copy(as_hbm, as_v)
        pltpu.sync_copy(ad_hbm, ad_v)

        @pl.loop(0, ZB, step=LANES)
        def _(i):
            z_v[pl.ds(i, LANES)] = jnp.zeros((LANES,), _F32)

        pltpu.sync_copy(z_v, den_sh.at[pl.ds(s * ZB, ZB)])
        plsc.subcore_barrier()

        def issue_load(k, b):
            base = base_w + k * CH
            pltpu.async_copy(src_hbm.at[pl.ds(base, CH)], src_v.at[b], sl[b])
            pltpu.async_copy(dst_hbm.at[pl.ds(base, CH)], dst_v.at[b], sl[b])

        def wait_load(k, b):
            base = base_w + k * CH
            pltpu.make_async_copy(src_hbm.at[pl.ds(base, CH)], src_v.at[b],
                                  sl[b]).wait()
            pltpu.make_async_copy(dst_hbm.at[pl.ds(base, CH)], dst_v.at[b],
                                  sl[b]).wait()

        def wait_out(k, b):
            base = base_w + k * CH
            pltpu.make_async_copy(p_v.at[b], p_hbm.at[pl.ds(base, CH)],
                                  so[b]).wait()

        def compute(b):
            @pl.loop(0, CH, step=LANES)
            def _(g):
                si = src_v[b, pl.ds(g, LANES)]
                di = dst_v[b, pl.ds(g, LANES)]
                e = plsc.load_gather(as_v, [si]) + plsc.load_gather(ad_v, [di])
                e = jnp.where(e >= 0.0, e, 0.2 * e)
                p_v[b, pl.ds(g, LANES)] = jnp.exp(e)

        def step(k, b):
            @pl.when(k > 1)
            def _():
                wait_out(k - 2, b)

            compute(b)
            base = base_w + k * CH
            pltpu.async_copy(p_v.at[b], p_hbm.at[pl.ds(base, CH)], so[b])
            pltpu.sync_copy(p_v.at[b], den_sh.at[dst_v.at[b]], add=True)

            @pl.when(k + 2 < NK)
            def _():
                issue_load(k + 2, b)

            @pl.when(k + 1 < NK)
            def _():
                wait_load(k + 1, 1 - b)

        pltpu.sync_copy(src_hbm.at[pl.ds(base_w, CH)], src_v.at[0])
        pltpu.sync_copy(dst_hbm.at[pl.ds(base_w, CH)], dst_v.at[0])
        issue_load(1, 1)

        @pl.loop(0, NK // 2)
        def _(kp):
            step(2 * kp, 0)
            step(2 * kp + 1, 1)

        wait_out(NK - 2, 0)
        wait_out(NK - 1, 1)

        rbase = base_w + NK * CH
        pltpu.sync_copy(src_hbm.at[pl.ds(rbase, REM)],
                        src_v.at[0, pl.ds(0, REM)])
        pltpu.sync_copy(dst_hbm.at[pl.ds(rbase, REM)],
                        dst_v.at[0, pl.ds(0, REM)])
        si = src_v[0, pl.ds(0, LANES)]
        di = dst_v[0, pl.ds(0, LANES)]
        rdst_v[...] = di
        e = plsc.load_gather(as_v, [si]) + plsc.load_gather(ad_v, [di])
        e = jnp.where(e >= 0.0, e, 0.2 * e)
        p_v[0, pl.ds(0, LANES)] = jnp.exp(e)
        pltpu.sync_copy(p_v.at[0, pl.ds(0, REM)], p_hbm.at[pl.ds(rbase, REM)])
        pltpu.sync_copy(p_v.at[0, pl.ds(0, REM)], den_sh.at[rdst_v], add=True)

        plsc.subcore_barrier()
        pltpu.sync_copy(den_sh.at[pl.ds(s * ZB, ZB)],
                        den_hbm.at[c, pl.ds(s * ZB, ZB)])

    return k(src, dst, a_s, a_d)


def _sc_pass2(src, dst, p, invden, h):
    @functools.partial(
        pl.kernel,
        out_type=jax.ShapeDtypeStruct((NC, NPAD, H), _F32),
        mesh=_mesh(),
        scratch_types=[
            pltpu.VMEM((NPAD,), _F32),
            pltpu.VMEM((2, CH), jnp.int32),
            pltpu.VMEM((2, CH), jnp.int32),
            pltpu.VMEM((2, 2, CH // 2), jnp.int32),
            pltpu.VMEM((2, CH), _F32),
            pltpu.VMEM((REM,), jnp.int32),
            pltpu.VMEM((CH, H), _F32),
            pltpu.VMEM_SHARED((NPAD, H), _F32),
            pltpu.SemaphoreType.DMA,
            pltpu.SemaphoreType.DMA,
            pltpu.SemaphoreType.DMA,
            pltpu.SemaphoreType.DMA,
            pltpu.SemaphoreType.DMA,
            pltpu.SemaphoreType.DMA,
        ],
        compiler_params=_SC_PARAMS,
    )
    def k(src_hbm, dst_hbm, p_hbm, invden_hbm, h_hbm, out_hbm,
          invden_v, src_v, dst_v, sdst_v, p_v, rdst_v, rows_v, out_sh,
          sl0, sl1, sga, sgb, ssa, ssb):
        c = jax.lax.axis_index("c")
        s = jax.lax.axis_index("s")
        w = _worker_id()
        base_w = w * PT
        sl = (sl0, sl1)
        pltpu.sync_copy(invden_hbm, invden_v)

        @pl.loop(0, CH)
        def _(r):
            @pl.loop(0, H, step=LANES)
            def _(j):
                rows_v[r, pl.ds(j, LANES)] = jnp.zeros((LANES,), _F32)

        @pl.loop(0, ZB // CH)
        def _(j):
            pltpu.sync_copy(rows_v, out_sh.at[pl.ds(s * ZB + j * CH, CH)])

        plsc.subcore_barrier()

        def issue_load(k, b):
            base = base_w + k * CH
            pltpu.async_copy(src_hbm.at[pl.ds(base, CH)], src_v.at[b], sl[b])
            pltpu.async_copy(dst_hbm.at[pl.ds(base, CH)], dst_v.at[b], sl[b])
            pltpu.async_copy(p_hbm.at[pl.ds(base, CH)], p_v.at[b], sl[b])

        def wait_load(k, b):
            base = base_w + k * CH
            pltpu.make_async_copy(src_hbm.at[pl.ds(base, CH)], src_v.at[b],
                                  sl[b]).wait()
            pltpu.make_async_copy(dst_hbm.at[pl.ds(base, CH)], dst_v.at[b],
                                  sl[b]).wait()
            pltpu.make_async_copy(p_hbm.at[pl.ds(base, CH)], p_v.at[b],
                                  sl[b]).wait()

        HF = CH // 2

        def scale_half(b, half):
            @pl.loop(0, HF, step=LANES)
            def _(g):
                w16 = p_v[b, pl.ds(half * HF + g, LANES)]
                for j in range(LANES):
                    wj = w16[j]
                    for cb in range(H // LANES):
                        cs = pl.ds(cb * LANES, LANES)
                        r = half * HF + g + j
                        rows_v[r, cs] = rows_v[r, cs] * wj

        def step(k, b):
            ga = pltpu.async_copy(h_hbm.at[src_v.at[b, pl.ds(0, HF)]],
                                  rows_v.at[pl.ds(0, HF)], sga)
            gb = pltpu.async_copy(h_hbm.at[src_v.at[b, pl.ds(HF, HF)]],
                                  rows_v.at[pl.ds(HF, HF)], sgb)
            for g in range(0, CH, LANES):
                di = dst_v[b, pl.ds(g, LANES)]
                sdst_v[b, (g // HF), pl.ds(g % HF, LANES)] = di
                p_v[b, pl.ds(g, LANES)] = (p_v[b, pl.ds(g, LANES)]
                                           * plsc.load_gather(invden_v, [di]))

            ga.wait()
            scale_half(b, 0)
            gb.wait()
            scale_half(b, 1)
            sa = pltpu.async_copy(rows_v.at[pl.ds(0, HF)],
                                  out_sh.at[sdst_v.at[b, 0]], ssa, add=True)
            sb = pltpu.async_copy(rows_v.at[pl.ds(HF, HF)],
                                  out_sh.at[sdst_v.at[b, 1]], ssb, add=True)

            @pl.when(k + 2 < NK)
            def _():
                issue_load(k + 2, b)

            @pl.when(k + 1 < NK)
            def _():
                wait_load(k + 1, 1 - b)

            sa.wait()
            sb.wait()

        pltpu.sync_copy(src_hbm.at[pl.ds(base_w, CH)], src_v.at[0])
        pltpu.sync_copy(dst_hbm.at[pl.ds(base_w, CH)], dst_v.at[0])
        pltpu.sync_copy(p_hbm.at[pl.ds(base_w, CH)], p_v.at[0])
        issue_load(1, 1)

        @pl.loop(0, NK // 2)
        def _(kp):
            step(2 * kp, 0)
            step(2 * kp + 1, 1)

        rbase = base_w + NK * CH
        pltpu.sync_copy(src_hbm.at[pl.ds(rbase, REM)],
                        src_v.at[0, pl.ds(0, REM)])
        pltpu.sync_copy(dst_hbm.at[pl.ds(rbase, REM)],
                        dst_v.at[0, pl.ds(0, REM)])
        pltpu.sync_copy(p_hbm.at[pl.ds(rbase, REM)], p_v.at[0, pl.ds(0, REM)])
        rsi = src_v[0, pl.ds(0, LANES)]
        rdi = dst_v[0, pl.ds(0, LANES)]
        rdst_v[...] = rsi
        pltpu.sync_copy(h_hbm.at[rdst_v], rows_v.at[pl.ds(0, REM)])
        rdst_v[...] = rdi
        w16 = p_v[0, pl.ds(0, LANES)] * plsc.load_gather(invden_v, [rdi])
        for j in range(LANES):
            wj = w16[j]
            for cb in range(H // LANES):
                cs = pl.ds(cb * LANES, LANES)
                rows_v[j, cs] = rows_v[j, cs] * wj
        pltpu.sync_copy(rows_v.at[pl.ds(0, REM)], out_sh.at[rdst_v],
                        add=True)

        plsc.subcore_barrier()

        @pl.loop(0, ZB // CH)
        def _(j):
            pltpu.sync_copy(out_sh.at[pl.ds(s * ZB + j * CH, CH)],
                            out_hbm.at[c, pl.ds(s * ZB + j * CH, CH)])

    return k(src, dst, p, invden, h)


def _gat_edges(src, dst, h, a_s, a_d):
    p, den = _sc_pass1(src, dst, a_s.reshape(N), a_d.reshape(N))
    invden = _tc_invden(den).reshape(NPAD)
    return _sc_pass2(src, dst, p, invden, h)


def kernel(x, edge_index, W_enc, a_src_enc, a_dst_enc, b_enc,
           W_proc, a_src_proc, a_dst_proc, b_proc,
           W_dec, a_src_dec, a_dst_dec, b_dec):
    ase, ade = a_src_enc.reshape(1, H), a_dst_enc.reshape(1, H)
    asp, adp = a_src_proc.reshape(1, H), a_dst_proc.reshape(1, H)
    asd, add = a_src_dec.reshape(1, H), a_dst_dec.reshape(1, H)
    be, bp, bd = b_enc.reshape(1, H), b_proc.reshape(1, H), b_dec.reshape(1, H)

    src, dst = edge_index[0], edge_index[1]
    h1, as1, ad1 = _tc_head(x, W_enc, ase, ade)
    parts1 = _gat_edges(src, dst, h1, as1, ad1)
    enc, h2, as2, ad2 = _tc_mid_self(parts1, be, W_proc, asp, adp)
    parts2 = _gat_edges(src, dst, h2, as2, ad2)
    _, h3, as3, ad3 = _tc_mid(parts2, bp, enc, W_proc, asp, adp)
    parts3 = _gat_edges(src, dst, h3, as3, ad3)
    h4, as4, ad4 = _tc_mid_single(parts3, bp, W_dec, asd, add)
    parts4 = _gat_edges(src, dst, h4, as4, ad4)
    return _tc_tail(parts4, bd)


# speedup vs baseline: 40.2748x; 1.1171x over previous
import functools

import jax
import jax.numpy as jnp
from jax.experimental import pallas as pl
from jax.experimental.pallas import tpu as pltpu
from jax.experimental.pallas import tpu_sc as plsc

N = 10000
E = 320000
H = 128
NC = 2
NS = 16
NW = NC * NS
LANES = 16
NPAD = 10240
ZB = NPAD // NS
CH = 128
NCHUNKS = E // CH
R = 2000

_F32 = jnp.float32


def _attn(h, asv_ref, adv_ref, as_ref, ad_ref):
    as_ref[...] = jnp.sum(h * asv_ref[...], axis=1)[:, None]
    ad_ref[...] = jnp.sum(h * adv_ref[...], axis=1)[:, None]


def _head_body(x_ref, w_ref, asv_ref, adv_ref, h_ref, as_ref, ad_ref):
    h = jnp.dot(x_ref[...], w_ref[...], preferred_element_type=_F32)
    h_ref[...] = h
    _attn(h, asv_ref, adv_ref, as_ref, ad_ref)


def _mid_self_body(parts_ref, b_ref, w_ref, asv_ref, adv_ref,
                   f_ref, h_ref, as_ref, ad_ref):
    f = parts_ref[0] + parts_ref[1] + b_ref[...]
    f_ref[...] = f
    ws = w_ref[0:H, :] + w_ref[H:2 * H, :]
    h = jnp.dot(f, ws, preferred_element_type=_F32)
    h_ref[...] = h
    _attn(h, asv_ref, adv_ref, as_ref, ad_ref)


def _mid_body(parts_ref, b_ref, g_ref, w_ref, asv_ref, adv_ref,
              f_ref, h_ref, as_ref, ad_ref):
    f = parts_ref[0] + parts_ref[1] + b_ref[...]
    f_ref[...] = f
    h = (jnp.dot(f, w_ref[0:H, :], preferred_element_type=_F32)
         + jnp.dot(g_ref[...], w_ref[H:2 * H, :], preferred_element_type=_F32))
    h_ref[...] = h
    _attn(h, asv_ref, adv_ref, as_ref, ad_ref)


def _mid_single_body(parts_ref, b_ref, w_ref, asv_ref, adv_ref,
                     h_ref, as_ref, ad_ref):
    f = parts_ref[0] + parts_ref[1] + b_ref[...]
    h = jnp.dot(f, w_ref[...], preferred_element_type=_F32)
    h_ref[...] = h
    _attn(h, asv_ref, adv_ref, as_ref, ad_ref)


def _tail_body(parts_ref, b_ref, out_ref):
    out_ref[...] = parts_ref[0] + parts_ref[1] + b_ref[...]


def _invden_body(den_ref, out_ref):
    out_ref[...] = (1.0 / (den_ref[0] + den_ref[1] + 1e-16))[None, :]


_vec_spec = pl.BlockSpec((1, H), lambda i: (0, 0))
_w1_spec = pl.BlockSpec((H, H), lambda i: (0, 0))
_w2_spec = pl.BlockSpec((2 * H, H), lambda i: (0, 0))
_row_spec = pl.BlockSpec((R, H), lambda i: (i, 0))
_parts_spec = pl.BlockSpec((2, R, H), lambda i: (0, i, 0))
_alpha_spec = pl.BlockSpec((R, 1), lambda i: (i, 0))

_h_shape = jax.ShapeDtypeStruct((N, H), _F32)
_alpha_shape = jax.ShapeDtypeStruct((N, 1), _F32)


def _tc_head(x, w, asv, adv):
    return pl.pallas_call(
        _head_body,
        grid=(N // R,),
        in_specs=[_row_spec, _w1_spec, _vec_spec, _vec_spec],
        out_specs=[_row_spec, _alpha_spec, _alpha_spec],
        out_shape=[_h_shape, _alpha_shape, _alpha_shape],
    )(x, w, asv, adv)


def _tc_mid_self(parts, b, w, asv, adv):
    return pl.pallas_call(
        _mid_self_body,
        grid=(N // R,),
        in_specs=[_parts_spec, _vec_spec, _w2_spec, _vec_spec, _vec_spec],
        out_specs=[_row_spec, _row_spec, _alpha_spec, _alpha_spec],
        out_shape=[_h_shape, _h_shape, _alpha_shape, _alpha_shape],
    )(parts, b, w, asv, adv)


def _tc_mid(parts, b, g, w, asv, adv):
    return pl.pallas_call(
        _mid_body,
        grid=(N // R,),
        in_specs=[_parts_spec, _vec_spec, _row_spec, _w2_spec, _vec_spec,
                  _vec_spec],
        out_specs=[_row_spec, _row_spec, _alpha_spec, _alpha_spec],
        out_shape=[_h_shape, _h_shape, _alpha_shape, _alpha_shape],
    )(parts, b, g, w, asv, adv)


def _tc_mid_single(parts, b, w, asv, adv):
    return pl.pallas_call(
        _mid_single_body,
        grid=(N // R,),
        in_specs=[_parts_spec, _vec_spec, _w1_spec, _vec_spec, _vec_spec],
        out_specs=[_row_spec, _alpha_spec, _alpha_spec],
        out_shape=[_h_shape, _alpha_shape, _alpha_shape],
    )(parts, b, w, asv, adv)


def _tc_invden(den):
    return pl.pallas_call(
        _invden_body,
        in_specs=[pl.BlockSpec((2, NPAD), lambda: (0, 0))],
        out_specs=pl.BlockSpec((1, NPAD), lambda: (0, 0)),
        out_shape=jax.ShapeDtypeStruct((1, NPAD), _F32),
    )(den)


def _tc_tail(parts, b):
    return pl.pallas_call(
        _tail_body,
        grid=(N // R,),
        in_specs=[_parts_spec, _vec_spec],
        out_specs=_row_spec,
        out_shape=_h_shape,
    )(parts, b)


def _mesh():
    return plsc.VectorSubcoreMesh(core_axis_name="c", subcore_axis_name="s")


_SC_PARAMS = pltpu.CompilerParams(needs_layout_passes=False)


def _worker_id():
    return jax.lax.axis_index("s") * NC + jax.lax.axis_index("c")


PT = E // NW
NK = PT // CH
REM = PT - NK * CH
SW = 128
CB = 2 * SW


def _sc_pass1(src, dst, a_s, a_d):
    @functools.partial(
        pl.kernel,
        out_type=(jax.ShapeDtypeStruct((E,), _F32),
                  jax.ShapeDtypeStruct((NC, NPAD), _F32)),
        mesh=_mesh(),
        scratch_types=[
            pltpu.VMEM((N,), _F32),
            pltpu.VMEM((N,), _F32),
            pltpu.VMEM((2, CH), jnp.int32),
            pltpu.VMEM((2, CH), jnp.int32),
            pltpu.VMEM((2, CH), _F32),
            pltpu.VMEM((ZB,), _F32),
            pltpu.VMEM((REM,), jnp.int32),
            pltpu.VMEM_SHARED((NPAD,), _F32),
            pltpu.SemaphoreType.DMA,
            pltpu.SemaphoreType.DMA,
            pltpu.SemaphoreType.DMA,
            pltpu.SemaphoreType.DMA,
        ],
        compiler_params=_SC_PARAMS,
    )
    def k(src_hbm, dst_hbm, as_hbm, ad_hbm, p_hbm, den_hbm,
          as_v, ad_v, src_v, dst_v, p_v, z_v, rdst_v, den_sh,
          sl0, sl1, so0, so1):
        c = jax.lax.axis_index("c")
        s = jax.lax.axis_index("s")
        w = _worker_id()
        base_w = w * PT
        sl = (sl0, sl1)
        so = (so0, so1)
        pltpu.sync_copy(as_hbm, as_v)
        pltpu.sync_copy(ad_hbm, ad_v)

        @pl.loop(0, ZB, step=LANES)
        def _(i):
            z_v[pl.ds(i, LANES)] = jnp.zeros((LANES,), _F32)

        pltpu.sync_copy(z_v, den_sh.at[pl.ds(s * ZB, ZB)])
        plsc.subcore_barrier()

        def issue_load(k, b):
            base = base_w + k * CH
            pltpu.async_copy(src_hbm.at[pl.ds(base, CH)], src_v.at[b], sl[b])
            pltpu.async_copy(dst_hbm.at[pl.ds(base, CH)], dst_v.at[b], sl[b])

        def wait_load(k, b):
            base = base_w + k * CH
            pltpu.make_async_copy(src_hbm.at[pl.ds(base, CH)], src_v.at[b],
                                  sl[b]).wait()
            pltpu.make_async_copy(dst_hbm.at[pl.ds(base, CH)], dst_v.at[b],
                                  sl[b]).wait()

        def wait_out(k, b):
            base = base_w + k * CH
            pltpu.make_async_copy(p_v.at[b], p_hbm.at[pl.ds(base, CH)],
                                  so[b]).wait()

        def compute(b):
            @pl.loop(0, CH, step=LANES)
            def _(g):
                si = src_v[b, pl.ds(g, LANES)]
                di = dst_v[b, pl.ds(g, LANES)]
                e = plsc.load_gather(as_v, [si]) + plsc.load_gather(ad_v, [di])
                e = jnp.where(e >= 0.0, e, 0.2 * e)
                p_v[b, pl.ds(g, LANES)] = jnp.exp(e)

        def step(k, b):
            @pl.when(k > 1)
            def _():
                wait_out(k - 2, b)

            compute(b)
            base = base_w + k * CH
            pltpu.async_copy(p_v.at[b], p_hbm.at[pl.ds(base, CH)], so[b])
            pltpu.sync_copy(p_v.at[b], den_sh.at[dst_v.at[b]], add=True)

            @pl.when(k + 2 < NK)
            def _():
                issue_load(k + 2, b)

            @pl.when(k + 1 < NK)
            def _():
                wait_load(k + 1, 1 - b)

        pltpu.sync_copy(src_hbm.at[pl.ds(base_w, CH)], src_v.at[0])
        pltpu.sync_copy(dst_hbm.at[pl.ds(base_w, CH)], dst_v.at[0])
        issue_load(1, 1)

        @pl.loop(0, NK // 2)
        def _(kp):
            step(2 * kp, 0)
            step(2 * kp + 1, 1)

        wait_out(NK - 2, 0)
        wait_out(NK - 1, 1)

        rbase = base_w + NK * CH
        pltpu.sync_copy(src_hbm.at[pl.ds(rbase, REM)],
                        src_v.at[0, pl.ds(0, REM)])
        pltpu.sync_copy(dst_hbm.at[pl.ds(rbase, REM)],
                        dst_v.at[0, pl.ds(0, REM)])
        si = src_v[0, pl.ds(0, LANES)]
        di = dst_v[0, pl.ds(0, LANES)]
        rdst_v[...] = di
        e = plsc.load_gather(as_v, [si]) + plsc.load_gather(ad_v, [di])
        e = jnp.where(e >= 0.0, e, 0.2 * e)
        p_v[0, pl.ds(0, LANES)] = jnp.exp(e)
        pltpu.sync_copy(p_v.at[0, pl.ds(0, REM)], p_hbm.at[pl.ds(rbase, REM)])
        pltpu.sync_copy(p_v.at[0, pl.ds(0, REM)], den_sh.at[rdst_v], add=True)

        plsc.subcore_barrier()
        pltpu.sync_copy(den_sh.at[pl.ds(s * ZB, ZB)],
                        den_hbm.at[c, pl.ds(s * ZB, ZB)])

    return k(src, dst, a_s, a_d)


def _sc_pass2(src, dst, p, invden, h):
    NKB = PT // CB
    HALF = NKB // 2

    @functools.partial(
        pl.kernel,
        out_type=jax.ShapeDtypeStruct((NC, NPAD, H), _F32),
        mesh=_mesh(),
        scratch_types=[
            pltpu.VMEM((N,), _F32),
            pltpu.VMEM((2, 2, SW), jnp.int32),
            pltpu.VMEM((2, 2, SW), jnp.int32),
            pltpu.VMEM((2, 2, SW), jnp.int32),
            pltpu.VMEM((2, 2, SW), _F32),
            pltpu.VMEM((REM,), jnp.int32),
            pltpu.VMEM((CB, H), _F32),
            pltpu.VMEM_SHARED((NPAD, H), _F32),
            pltpu.SemaphoreType.DMA,
            pltpu.SemaphoreType.DMA,
            pltpu.SemaphoreType.DMA,
            pltpu.SemaphoreType.DMA,
            pltpu.SemaphoreType.DMA,
            pltpu.SemaphoreType.DMA,
        ],
        compiler_params=_SC_PARAMS,
    )
    def k(src_hbm, dst_hbm, p_hbm, invden_hbm, h_hbm, out_hbm,
          invden_v, src_v, dst_v, sdst_v, p_v, rdst_v, rows_v, out_sh,
          sl0, sl1, sga, sgb, ssa, ssb):
        c = jax.lax.axis_index("c")
        s = jax.lax.axis_index("s")
        w = _worker_id()
        base_w = w * PT
        sl = (sl0, sl1)
        pltpu.sync_copy(invden_hbm.at[pl.ds(0, N)], invden_v)

        @pl.loop(0, CH)
        def _(r):
            @pl.loop(0, H, step=LANES)
            def _(j):
                rows_v[r, pl.ds(j, LANES)] = jnp.zeros((LANES,), _F32)

        @pl.loop(0, ZB // CH)
        def _(j):
            pltpu.sync_copy(rows_v.at[pl.ds(0, CH)],
                            out_sh.at[pl.ds(s * ZB + j * CH, CH)])

        plsc.subcore_barrier()

        def issue_load(k, b):
            for hh in range(2):
                base = base_w + k * CB + hh * SW
                pltpu.async_copy(src_hbm.at[pl.ds(base, SW)],
                                 src_v.at[b, hh], sl[b])
                pltpu.async_copy(dst_hbm.at[pl.ds(base, SW)],
                                 dst_v.at[b, hh], sl[b])
                pltpu.async_copy(p_hbm.at[pl.ds(base, SW)],
                                 p_v.at[b, hh], sl[b])

        def wait_load(k, b):
            for hh in range(2):
                base = base_w + k * CB + hh * SW
                pltpu.make_async_copy(src_hbm.at[pl.ds(base, SW)],
                                      src_v.at[b, hh], sl[b]).wait()
                pltpu.make_async_copy(dst_hbm.at[pl.ds(base, SW)],
                                      dst_v.at[b, hh], sl[b]).wait()
                pltpu.make_async_copy(p_hbm.at[pl.ds(base, SW)],
                                      p_v.at[b, hh], sl[b]).wait()

        def scale_half(b, half):
            @pl.loop(0, SW, step=LANES)
            def _(g):
                w16 = p_v[b, half, pl.ds(g, LANES)]
                for j in range(LANES):
                    wj = w16[j]
                    for cb in range(H // LANES):
                        cs = pl.ds(cb * LANES, LANES)
                        r = half * SW + g + j
                        rows_v[r, cs] = rows_v[r, cs] * wj

        def step(k, b):
            ga = pltpu.async_copy(h_hbm.at[src_v.at[b, 0]],
                                  rows_v.at[pl.ds(0, SW)], sga)
            gb = pltpu.async_copy(h_hbm.at[src_v.at[b, 1]],
                                  rows_v.at[pl.ds(SW, SW)], sgb)
            for hh in range(2):
                for g in range(0, SW, LANES):
                    di = dst_v[b, hh, pl.ds(g, LANES)]
                    sdst_v[b, hh, pl.ds(g, LANES)] = di
                    p_v[b, hh, pl.ds(g, LANES)] = (
                        p_v[b, hh, pl.ds(g, LANES)]
                        * plsc.load_gather(invden_v, [di]))

            ga.wait()
            scale_half(b, 0)
            sa = pltpu.async_copy(rows_v.at[pl.ds(0, SW)],
                                  out_sh.at[sdst_v.at[b, 0]], ssa, add=True)
            gb.wait()
            scale_half(b, 1)
            sb = pltpu.async_copy(rows_v.at[pl.ds(SW, SW)],
                                  out_sh.at[sdst_v.at[b, 1]], ssb, add=True)

            @pl.when(k + 2 < NKB)
            def _():
                issue_load(k + 2, b)

            @pl.when(k + 1 < NKB)
            def _():
                wait_load(k + 1, 1 - b)

            sa.wait()
            sb.wait()

        for hh in range(2):
            pbase = base_w + hh * SW
            pltpu.sync_copy(src_hbm.at[pl.ds(pbase, SW)], src_v.at[0, hh])
            pltpu.sync_copy(dst_hbm.at[pl.ds(pbase, SW)], dst_v.at[0, hh])
            pltpu.sync_copy(p_hbm.at[pl.ds(pbase, SW)], p_v.at[0, hh])
        issue_load(1, 1)

        @pl.loop(0, HALF)
        def _(kp):
            step(2 * kp, 0)
            step(2 * kp + 1, 1)

        step(NKB - 1, 0)

        rbase = base_w + NKB * CB
        pltpu.sync_copy(src_hbm.at[pl.ds(rbase, REM)],
                        src_v.at[0, 0, pl.ds(0, REM)])
        pltpu.sync_copy(dst_hbm.at[pl.ds(rbase, REM)],
                        dst_v.at[0, 0, pl.ds(0, REM)])
        pltpu.sync_copy(p_hbm.at[pl.ds(rbase, REM)],
                        p_v.at[0, 0, pl.ds(0, REM)])
        rsi = src_v[0, 0, pl.ds(0, LANES)]
        rdi = dst_v[0, 0, pl.ds(0, LANES)]
        rdst_v[...] = rsi
        pltpu.sync_copy(h_hbm.at[rdst_v], rows_v.at[pl.ds(0, REM)])
        rdst_v[...] = rdi
        w16 = p_v[0, 0, pl.ds(0, LANES)] * plsc.load_gather(invden_v, [rdi])
        for j in range(LANES):
            wj = w16[j]
            for cb in range(H // LANES):
                cs = pl.ds(cb * LANES, LANES)
                rows_v[j, cs] = rows_v[j, cs] * wj
        pltpu.sync_copy(rows_v.at[pl.ds(0, REM)], out_sh.at[rdst_v],
                        add=True)

        plsc.subcore_barrier()

        @pl.loop(0, ZB // CH)
        def _(j):
            pltpu.sync_copy(out_sh.at[pl.ds(s * ZB + j * CH, CH)],
                            out_hbm.at[c, pl.ds(s * ZB + j * CH, CH)])

    return k(src, dst, p, invden, h)


def _gat_edges(src, dst, h, a_s, a_d):
    p, den = _sc_pass1(src, dst, a_s.reshape(N), a_d.reshape(N))
    invden = _tc_invden(den).reshape(NPAD)
    return _sc_pass2(src, dst, p, invden, h)


def kernel(x, edge_index, W_enc, a_src_enc, a_dst_enc, b_enc,
           W_proc, a_src_proc, a_dst_proc, b_proc,
           W_dec, a_src_dec, a_dst_dec, b_dec):
    ase, ade = a_src_enc.reshape(1, H), a_dst_enc.reshape(1, H)
    asp, adp = a_src_proc.reshape(1, H), a_dst_proc.reshape(1, H)
    asd, add = a_src_dec.reshape(1, H), a_dst_dec.reshape(1, H)
    be, bp, bd = b_enc.reshape(1, H), b_proc.reshape(1, H), b_dec.reshape(1, H)

    src, dst = edge_index[0], edge_index[1]
    h1, as1, ad1 = _tc_head(x, W_enc, ase, ade)
    parts1 = _gat_edges(src, dst, h1, as1, ad1)
    enc, h2, as2, ad2 = _tc_mid_self(parts1, be, W_proc, asp, adp)
    parts2 = _gat_edges(src, dst, h2, as2, ad2)
    _, h3, as3, ad3 = _tc_mid(parts2, bp, enc, W_proc, asp, adp)
    parts3 = _gat_edges(src, dst, h3, as3, ad3)
    h4, as4, ad4 = _tc_mid_single(parts3, bp, W_dec, asd, add)
    parts4 = _gat_edges(src, dst, h4, as4, ad4)
    return _tc_tail(parts4, bd)


# speedup vs baseline: 45.3967x; 1.1272x over previous
import functools

import jax
import jax.numpy as jnp
from jax.experimental import pallas as pl
from jax.experimental.pallas import tpu as pltpu
from jax.experimental.pallas import tpu_sc as plsc

N = 10000
E = 320000
H = 128
NC = 2
NS = 16
NW = NC * NS
LANES = 16
NPAD = 10240
ZB = NPAD // NS
CH = 128
NCHUNKS = E // CH
R = 2000

_F32 = jnp.float32


def _attn(h, asv_ref, adv_ref, as_ref, ad_ref):
    as_ref[...] = jnp.sum(h * asv_ref[...], axis=1)[:, None]
    ad_ref[...] = jnp.sum(h * adv_ref[...], axis=1)[:, None]


def _head_body(x_ref, w_ref, asv_ref, adv_ref, h_ref, as_ref, ad_ref):
    h = jnp.dot(x_ref[...], w_ref[...], preferred_element_type=_F32)
    h_ref[...] = h
    _attn(h, asv_ref, adv_ref, as_ref, ad_ref)


def _mid_self_body(parts_ref, b_ref, w_ref, asv_ref, adv_ref,
                   f_ref, h_ref, as_ref, ad_ref):
    f = parts_ref[0] + parts_ref[1] + b_ref[...]
    f_ref[...] = f
    ws = w_ref[0:H, :] + w_ref[H:2 * H, :]
    h = jnp.dot(f, ws, preferred_element_type=_F32)
    h_ref[...] = h
    _attn(h, asv_ref, adv_ref, as_ref, ad_ref)


def _mid_body(parts_ref, b_ref, g_ref, w_ref, asv_ref, adv_ref,
              f_ref, h_ref, as_ref, ad_ref):
    f = parts_ref[0] + parts_ref[1] + b_ref[...]
    f_ref[...] = f
    h = (jnp.dot(f, w_ref[0:H, :], preferred_element_type=_F32)
         + jnp.dot(g_ref[...], w_ref[H:2 * H, :], preferred_element_type=_F32))
    h_ref[...] = h
    _attn(h, asv_ref, adv_ref, as_ref, ad_ref)


def _mid_single_body(parts_ref, b_ref, w_ref, asv_ref, adv_ref,
                     h_ref, as_ref, ad_ref):
    f = parts_ref[0] + parts_ref[1] + b_ref[...]
    h = jnp.dot(f, w_ref[...], preferred_element_type=_F32)
    h_ref[...] = h
    _attn(h, asv_ref, adv_ref, as_ref, ad_ref)


def _tail_body(parts_ref, b_ref, out_ref):
    out_ref[...] = parts_ref[0] + parts_ref[1] + b_ref[...]


def _invden_body(den_ref, out_ref):
    out_ref[...] = (1.0 / (den_ref[0] + den_ref[1] + 1e-16))[None, :]


_vec_spec = pl.BlockSpec((1, H), lambda i: (0, 0))
_w1_spec = pl.BlockSpec((H, H), lambda i: (0, 0))
_w2_spec = pl.BlockSpec((2 * H, H), lambda i: (0, 0))
_row_spec = pl.BlockSpec((R, H), lambda i: (i, 0))
_parts_spec = pl.BlockSpec((2, R, H), lambda i: (0, i, 0))
_alpha_spec = pl.BlockSpec((R, 1), lambda i: (i, 0))

_h_shape = jax.ShapeDtypeStruct((N, H), _F32)
_alpha_shape = jax.ShapeDtypeStruct((N, 1), _F32)


def _tc_head(x, w, asv, adv):
    return pl.pallas_call(
        _head_body,
        grid=(N // R,),
        in_specs=[_row_spec, _w1_spec, _vec_spec, _vec_spec],
        out_specs=[_row_spec, _alpha_spec, _alpha_spec],
        out_shape=[_h_shape, _alpha_shape, _alpha_shape],
    )(x, w, asv, adv)


def _tc_mid_self(parts, b, w, asv, adv):
    return pl.pallas_call(
        _mid_self_body,
        grid=(N // R,),
        in_specs=[_parts_spec, _vec_spec, _w2_spec, _vec_spec, _vec_spec],
        out_specs=[_row_spec, _row_spec, _alpha_spec, _alpha_spec],
        out_shape=[_h_shape, _h_shape, _alpha_shape, _alpha_shape],
    )(parts, b, w, asv, adv)


def _tc_mid(parts, b, g, w, asv, adv):
    return pl.pallas_call(
        _mid_body,
        grid=(N // R,),
        in_specs=[_parts_spec, _vec_spec, _row_spec, _w2_spec, _vec_spec,
                  _vec_spec],
        out_specs=[_row_spec, _row_spec, _alpha_spec, _alpha_spec],
        out_shape=[_h_shape, _h_shape, _alpha_shape, _alpha_shape],
    )(parts, b, g, w, asv, adv)


def _tc_mid_single(parts, b, w, asv, adv):
    return pl.pallas_call(
        _mid_single_body,
        grid=(N // R,),
        in_specs=[_parts_spec, _vec_spec, _w1_spec, _vec_spec, _vec_spec],
        out_specs=[_row_spec, _alpha_spec, _alpha_spec],
        out_shape=[_h_shape, _alpha_shape, _alpha_shape],
    )(parts, b, w, asv, adv)


def _tc_invden(den):
    return pl.pallas_call(
        _invden_body,
        in_specs=[pl.BlockSpec((2, NPAD), lambda: (0, 0))],
        out_specs=pl.BlockSpec((1, NPAD), lambda: (0, 0)),
        out_shape=jax.ShapeDtypeStruct((1, NPAD), _F32),
    )(den)


def _tc_tail(parts, b):
    return pl.pallas_call(
        _tail_body,
        grid=(N // R,),
        in_specs=[_parts_spec, _vec_spec],
        out_specs=_row_spec,
        out_shape=_h_shape,
    )(parts, b)


def _mesh():
    return plsc.VectorSubcoreMesh(core_axis_name="c", subcore_axis_name="s")


_SC_PARAMS = pltpu.CompilerParams(needs_layout_passes=False)


def _worker_id():
    return jax.lax.axis_index("s") * NC + jax.lax.axis_index("c")


PT = E // NW
NK = PT // CH
REM = PT - NK * CH
SW = 128
CB = 2 * SW


def _sc_pass1(src, dst, a_s, a_d):
    @functools.partial(
        pl.kernel,
        out_type=(jax.ShapeDtypeStruct((E,), _F32),
                  jax.ShapeDtypeStruct((NC, NPAD), _F32)),
        mesh=_mesh(),
        scratch_types=[
            pltpu.VMEM((N,), _F32),
            pltpu.VMEM((N,), _F32),
            pltpu.VMEM((2, CH), jnp.int32),
            pltpu.VMEM((2, CH), jnp.int32),
            pltpu.VMEM((2, CH), _F32),
            pltpu.VMEM((ZB,), _F32),
            pltpu.VMEM((REM,), jnp.int32),
            pltpu.VMEM_SHARED((NPAD,), _F32),
            pltpu.SemaphoreType.DMA,
            pltpu.SemaphoreType.DMA,
            pltpu.SemaphoreType.DMA,
            pltpu.SemaphoreType.DMA,
        ],
        compiler_params=_SC_PARAMS,
    )
    def k(src_hbm, dst_hbm, as_hbm, ad_hbm, p_hbm, den_hbm,
          as_v, ad_v, src_v, dst_v, p_v, z_v, rdst_v, den_sh,
          sl0, sl1, so0, so1):
        c = jax.lax.axis_index("c")
        s = jax.lax.axis_index("s")
        w = _worker_id()
        base_w = w * PT
        sl = (sl0, sl1)
        so = (so0, so1)
        pltpu.sync_copy(as_hbm, as_v)
        pltpu.sync_copy(ad_hbm, ad_v)

        @pl.loop(0, ZB, step=LANES)
        def _(i):
            z_v[pl.ds(i, LANES)] = jnp.zeros((LANES,), _F32)

        pltpu.sync_copy(z_v, den_sh.at[pl.ds(s * ZB, ZB)])
        plsc.subcore_barrier()

        def issue_load(k, b):
            base = base_w + k * CH
            pltpu.async_copy(src_hbm.at[pl.ds(base, CH)], src_v.at[b], sl[b])
            pltpu.async_copy(dst_hbm.at[pl.ds(base, CH)], dst_v.at[b], sl[b])

        def wait_load(k, b):
            base = base_w + k * CH
            pltpu.make_async_copy(src_hbm.at[pl.ds(base, CH)], src_v.at[b],
                                  sl[b]).wait()
            pltpu.make_async_copy(dst_hbm.at[pl.ds(base, CH)], dst_v.at[b],
                                  sl[b]).wait()

        def wait_out(k, b):
            base = base_w + k * CH
            pltpu.make_async_copy(p_v.at[b], p_hbm.at[pl.ds(base, CH)],
                                  so[b]).wait()

        def compute(b):
            @pl.loop(0, CH, step=LANES)
            def _(g):
                si = src_v[b, pl.ds(g, LANES)]
                di = dst_v[b, pl.ds(g, LANES)]
                e = plsc.load_gather(as_v, [si]) + plsc.load_gather(ad_v, [di])
                e = jnp.where(e >= 0.0, e, 0.2 * e)
                p_v[b, pl.ds(g, LANES)] = jnp.exp(e)

        def step(k, b):
            @pl.when(k > 1)
            def _():
                wait_out(k - 2, b)

            compute(b)
            base = base_w + k * CH
            pltpu.async_copy(p_v.at[b], p_hbm.at[pl.ds(base, CH)], so[b])
            pltpu.sync_copy(p_v.at[b], den_sh.at[dst_v.at[b]], add=True)

            @pl.when(k + 2 < NK)
            def _():
                issue_load(k + 2, b)

            @pl.when(k + 1 < NK)
            def _():
                wait_load(k + 1, 1 - b)

        pltpu.sync_copy(src_hbm.at[pl.ds(base_w, CH)], src_v.at[0])
        pltpu.sync_copy(dst_hbm.at[pl.ds(base_w, CH)], dst_v.at[0])
        issue_load(1, 1)

        @pl.loop(0, NK // 2)
        def _(kp):
            step(2 * kp, 0)
            step(2 * kp + 1, 1)

        wait_out(NK - 2, 0)
        wait_out(NK - 1, 1)

        rbase = base_w + NK * CH
        pltpu.sync_copy(src_hbm.at[pl.ds(rbase, REM)],
                        src_v.at[0, pl.ds(0, REM)])
        pltpu.sync_copy(dst_hbm.at[pl.ds(rbase, REM)],
                        dst_v.at[0, pl.ds(0, REM)])
        si = src_v[0, pl.ds(0, LANES)]
        di = dst_v[0, pl.ds(0, LANES)]
        rdst_v[...] = di
        e = plsc.load_gather(as_v, [si]) + plsc.load_gather(ad_v, [di])
        e = jnp.where(e >= 0.0, e, 0.2 * e)
        p_v[0, pl.ds(0, LANES)] = jnp.exp(e)
        pltpu.sync_copy(p_v.at[0, pl.ds(0, REM)], p_hbm.at[pl.ds(rbase, REM)])
        pltpu.sync_copy(p_v.at[0, pl.ds(0, REM)], den_sh.at[rdst_v], add=True)

        plsc.subcore_barrier()
        pltpu.sync_copy(den_sh.at[pl.ds(s * ZB, ZB)],
                        den_hbm.at[c, pl.ds(s * ZB, ZB)])

    return k(src, dst, a_s, a_d)


def _sc_pass2(src, dst, p, invden, h):
    NKB = PT // CB
    HALF = NKB // 2

    @functools.partial(
        pl.kernel,
        out_type=jax.ShapeDtypeStruct((NC, NPAD, H), _F32),
        mesh=_mesh(),
        scratch_types=[
            pltpu.VMEM((N,), _F32),
            pltpu.VMEM((2, 2, SW), jnp.int32),
            pltpu.VMEM((2, 2, SW), jnp.int32),
            pltpu.VMEM((2, 2, SW), jnp.int32),
            pltpu.VMEM((2, 2, SW), _F32),
            pltpu.VMEM((REM,), jnp.int32),
            pltpu.VMEM((CB, H), _F32),
            pltpu.VMEM_SHARED((NPAD, H), _F32),
            pltpu.SemaphoreType.DMA,
            pltpu.SemaphoreType.DMA,
            pltpu.SemaphoreType.DMA,
            pltpu.SemaphoreType.DMA,
            pltpu.SemaphoreType.DMA,
            pltpu.SemaphoreType.DMA,
        ],
        compiler_params=_SC_PARAMS,
    )
    def k(src_hbm, dst_hbm, p_hbm, invden_hbm, h_hbm, out_hbm,
          invden_v, src_v, dst_v, sdst_v, p_v, rdst_v, rows_v, out_sh,
          sl0, sl1, sga, sgb, ssa, ssb):
        c = jax.lax.axis_index("c")
        s = jax.lax.axis_index("s")
        w = _worker_id()
        base_w = w * PT
        sl = (sl0, sl1)
        pltpu.sync_copy(invden_hbm.at[pl.ds(0, N)], invden_v)

        @pl.loop(0, CH)
        def _(r):
            @pl.loop(0, H, step=LANES)
            def _(j):
                rows_v[r, pl.ds(j, LANES)] = jnp.zeros((LANES,), _F32)

        @pl.loop(0, ZB // CH)
        def _(j):
            pltpu.sync_copy(rows_v.at[pl.ds(0, CH)],
                            out_sh.at[pl.ds(s * ZB + j * CH, CH)])

        plsc.subcore_barrier()

        def issue_load(k, b):
            for hh in range(2):
                base = base_w + k * CB + hh * SW
                pltpu.async_copy(src_hbm.at[pl.ds(base, SW)],
                                 src_v.at[b, hh], sl[b])
                pltpu.async_copy(dst_hbm.at[pl.ds(base, SW)],
                                 dst_v.at[b, hh], sl[b])
                pltpu.async_copy(p_hbm.at[pl.ds(base, SW)],
                                 p_v.at[b, hh], sl[b])

        def wait_load(k, b):
            for hh in range(2):
                base = base_w + k * CB + hh * SW
                pltpu.make_async_copy(src_hbm.at[pl.ds(base, SW)],
                                      src_v.at[b, hh], sl[b]).wait()
                pltpu.make_async_copy(dst_hbm.at[pl.ds(base, SW)],
                                      dst_v.at[b, hh], sl[b]).wait()
                pltpu.make_async_copy(p_hbm.at[pl.ds(base, SW)],
                                      p_v.at[b, hh], sl[b]).wait()

        def scale_half(b, half):
            @pl.loop(0, SW, step=LANES)
            def _(g):
                w16 = p_v[b, half, pl.ds(g, LANES)]
                for j in range(LANES):
                    wj = w16[j]
                    for cb in range(H // LANES):
                        cs = pl.ds(cb * LANES, LANES)
                        r = half * SW + g + j
                        rows_v[r, cs] = rows_v[r, cs] * wj

        def step(k, b):
            ga = pltpu.async_copy(h_hbm.at[src_v.at[b, 0]],
                                  rows_v.at[pl.ds(0, SW)], sga)
            for hh in range(2):
                for g in range(0, SW, LANES):
                    di = dst_v[b, hh, pl.ds(g, LANES)]
                    sdst_v[b, hh, pl.ds(g, LANES)] = di
                    p_v[b, hh, pl.ds(g, LANES)] = (
                        p_v[b, hh, pl.ds(g, LANES)]
                        * plsc.load_gather(invden_v, [di]))

            @pl.when(k > 0)
            def _():
                pltpu.make_async_copy(rows_v.at[pl.ds(SW, SW)],
                                      out_sh.at[sdst_v.at[1 - b, 1]],
                                      ssb).wait()

            gb = pltpu.async_copy(h_hbm.at[src_v.at[b, 1]],
                                  rows_v.at[pl.ds(SW, SW)], sgb)
            ga.wait()
            scale_half(b, 0)
            sa = pltpu.async_copy(rows_v.at[pl.ds(0, SW)],
                                  out_sh.at[sdst_v.at[b, 0]], ssa, add=True)
            gb.wait()
            scale_half(b, 1)
            pltpu.async_copy(rows_v.at[pl.ds(SW, SW)],
                             out_sh.at[sdst_v.at[b, 1]], ssb, add=True)

            @pl.when(k + 2 < NKB)
            def _():
                issue_load(k + 2, b)

            @pl.when(k + 1 < NKB)
            def _():
                wait_load(k + 1, 1 - b)

            sa.wait()

        for hh in range(2):
            pbase = base_w + hh * SW
            pltpu.sync_copy(src_hbm.at[pl.ds(pbase, SW)], src_v.at[0, hh])
            pltpu.sync_copy(dst_hbm.at[pl.ds(pbase, SW)], dst_v.at[0, hh])
            pltpu.sync_copy(p_hbm.at[pl.ds(pbase, SW)], p_v.at[0, hh])
        issue_load(1, 1)

        @pl.loop(0, HALF)
        def _(kp):
            step(2 * kp, 0)
            step(2 * kp + 1, 1)

        step(NKB - 1, 0)
        pltpu.make_async_copy(rows_v.at[pl.ds(SW, SW)],
                              out_sh.at[sdst_v.at[0, 1]], ssb).wait()

        rbase = base_w + NKB * CB
        pltpu.sync_copy(src_hbm.at[pl.ds(rbase, REM)],
                        src_v.at[0, 0, pl.ds(0, REM)])
        pltpu.sync_copy(dst_hbm.at[pl.ds(rbase, REM)],
                        dst_v.at[0, 0, pl.ds(0, REM)])
        pltpu.sync_copy(p_hbm.at[pl.ds(rbase, REM)],
                        p_v.at[0, 0, pl.ds(0, REM)])
        rsi = src_v[0, 0, pl.ds(0, LANES)]
        rdi = dst_v[0, 0, pl.ds(0, LANES)]
        rdst_v[...] = rsi
        pltpu.sync_copy(h_hbm.at[rdst_v], rows_v.at[pl.ds(0, REM)])
        rdst_v[...] = rdi
        w16 = p_v[0, 0, pl.ds(0, LANES)] * plsc.load_gather(invden_v, [rdi])
        for j in range(LANES):
            wj = w16[j]
            for cb in range(H // LANES):
                cs = pl.ds(cb * LANES, LANES)
                rows_v[j, cs] = rows_v[j, cs] * wj
        pltpu.sync_copy(rows_v.at[pl.ds(0, REM)], out_sh.at[rdst_v],
                        add=True)

        plsc.subcore_barrier()

        @pl.loop(0, ZB // CH)
        def _(j):
            pltpu.sync_copy(out_sh.at[pl.ds(s * ZB + j * CH, CH)],
                            out_hbm.at[c, pl.ds(s * ZB + j * CH, CH)])

    return k(src, dst, p, invden, h)


def _gat_edges(src, dst, h, a_s, a_d):
    p, den = _sc_pass1(src, dst, a_s.reshape(N), a_d.reshape(N))
    invden = _tc_invden(den).reshape(NPAD)
    return _sc_pass2(src, dst, p, invden, h)


def kernel(x, edge_index, W_enc, a_src_enc, a_dst_enc, b_enc,
           W_proc, a_src_proc, a_dst_proc, b_proc,
           W_dec, a_src_dec, a_dst_dec, b_dec):
    ase, ade = a_src_enc.reshape(1, H), a_dst_enc.reshape(1, H)
    asp, adp = a_src_proc.reshape(1, H), a_dst_proc.reshape(1, H)
    asd, add = a_src_dec.reshape(1, H), a_dst_dec.reshape(1, H)
    be, bp, bd = b_enc.reshape(1, H), b_proc.reshape(1, H), b_dec.reshape(1, H)

    src, dst = edge_index[0], edge_index[1]
    h1, as1, ad1 = _tc_head(x, W_enc, ase, ade)
    parts1 = _gat_edges(src, dst, h1, as1, ad1)
    enc, h2, as2, ad2 = _tc_mid_self(parts1, be, W_proc, asp, adp)
    parts2 = _gat_edges(src, dst, h2, as2, ad2)
    _, h3, as3, ad3 = _tc_mid(parts2, bp, enc, W_proc, asp, adp)
    parts3 = _gat_edges(src, dst, h3, as3, ad3)
    h4, as4, ad4 = _tc_mid_single(parts3, bp, W_dec, asd, add)
    parts4 = _gat_edges(src, dst, h4, as4, ad4)
    return _tc_tail(parts4, bd)
